# Initial kernel scaffold; baseline (speedup 1.0000x reference)
import jax
import jax.numpy as jnp
from jax.experimental import pallas as pl


def kernel(x, edge_index, batch_size, batch_index, W1, b1, W2, b2, W3, b3, enc1_W, enc1_b, enc2_W, enc2_b, dec1_W, dec1_b, dec2_W, dec2_b):
    raise NotImplementedError("write your pallas kernel here")


# speedup vs baseline: 25.5829x; 25.5829x over previous
import functools

import jax
import jax.numpy as jnp
from jax import lax
from jax.experimental import pallas as pl
from jax.experimental.pallas import tpu as pltpu
from jax.experimental.pallas import tpu_sc as plsc

N = 10000
NPAD = 10240
E = 320000
F_IN = 128
NC = 2
NS = 16
NW = NC * NS
CHUNK = 128
EW = 10240
E_PAD = EW * NW
ROWS_W = EW // CHUNK
NODES_T = NPAD // NS

@functools.cache
def _mesh():
    return plsc.VectorSubcoreMesh(
        core_axis_name="c", subcore_axis_name="s",
        num_cores=NC, num_subcores=NS)


def _sc_degree(dst_rows, zz, ones):
    def body(dst_hbm, zz_hbm, ones_hbm, out_hbm, idx_v, ones_v, zz_v, acc_sh):
        cid = lax.axis_index("c")
        sid = lax.axis_index("s")
        w = sid * NC + cid
        pltpu.sync_copy(zz_hbm.at[pl.ds(sid * NODES_T, NODES_T)], zz_v)
        pltpu.sync_copy(zz_v, acc_sh.at[pl.ds(sid * NODES_T, NODES_T)])
        pltpu.sync_copy(ones_hbm, ones_v)
        plsc.subcore_barrier()
        pltpu.sync_copy(dst_hbm.at[pl.ds(w * ROWS_W, ROWS_W)], idx_v)

        def step(g, carry):
            pltpu.sync_copy(ones_v, acc_sh.at[idx_v.at[g]], add=True)
            return carry

        lax.fori_loop(0, ROWS_W, step, 0)
        plsc.subcore_barrier()
        pltpu.sync_copy(acc_sh.at[pl.ds(sid * NODES_T, NODES_T)],
                        out_hbm.at[cid, pl.ds(sid * NODES_T, NODES_T)])

    k = pl.kernel(
        body,
        out_type=jax.ShapeDtypeStruct((NC, NPAD), jnp.float32),
        mesh=_mesh(),
        compiler_params=pltpu.CompilerParams(use_tc_tiling_on_sc=False),
        scratch_types=[
            pltpu.VMEM((ROWS_W, CHUNK), jnp.int32),
            pltpu.VMEM((CHUNK,), jnp.float32),
            pltpu.VMEM((NODES_T,), jnp.float32),
            pltpu.VMEM_SHARED((NPAD,), jnp.float32),
        ],
    )
    return k(dst_rows, zz, ones)


def _sc_propagate(yw, src_rows, dst_rows, zz, F):
    def body(yw_hbm, src_hbm, dst_hbm, zz_hbm, out_hbm,
             sidx_v, didx_v, rows_v, zz_v, acc_sh, sem):
        cid = lax.axis_index("c")
        sid = lax.axis_index("s")
        w = sid * NC + cid
        pltpu.sync_copy(zz_hbm.at[pl.ds(sid * NODES_T, NODES_T)], zz_v)
        pltpu.sync_copy(zz_v, acc_sh.at[pl.ds(sid * NODES_T, NODES_T)])
        plsc.subcore_barrier()
        pltpu.sync_copy(src_hbm.at[pl.ds(w * ROWS_W, ROWS_W)], sidx_v)
        pltpu.sync_copy(dst_hbm.at[pl.ds(w * ROWS_W, ROWS_W)], didx_v)

        def step(g, carry):
            pltpu.async_copy(yw_hbm.at[sidx_v.at[g]], rows_v, sem).wait()
            pltpu.sync_copy(rows_v, acc_sh.at[didx_v.at[g]], add=True)
            return carry

        lax.fori_loop(0, ROWS_W, step, 0)
        plsc.subcore_barrier()
        pltpu.sync_copy(acc_sh.at[pl.ds(sid * NODES_T, NODES_T)],
                        out_hbm.at[cid, pl.ds(sid * NODES_T, NODES_T)])

    k = pl.kernel(
        body,
        out_type=jax.ShapeDtypeStruct((NC, NPAD, F), jnp.float32),
        mesh=_mesh(),
        compiler_params=pltpu.CompilerParams(use_tc_tiling_on_sc=False),
        scratch_types=[
            pltpu.VMEM((ROWS_W, CHUNK), jnp.int32),
            pltpu.VMEM((ROWS_W, CHUNK), jnp.int32),
            pltpu.VMEM((CHUNK, F), jnp.float32),
            pltpu.VMEM((NODES_T, F), jnp.float32),
            pltpu.VMEM_SHARED((NPAD, F), jnp.float32),
            pltpu.SemaphoreType.DMA,
        ],
    )
    return k(yw, src_rows, dst_rows, zz)


_BN = 1024


def _tc_prep(degT, x_pad, W1):
    fo = W1.shape[1]

    def body(deg_ref, x_ref, w_ref, dis_ref, yw_ref):
        d = lax.rsqrt(deg_ref[:, 0:1] + deg_ref[:, 1:2] + 1.0)
        dis_ref[...] = d
        yw_ref[...] = d * jnp.dot(x_ref[...], w_ref[...],
                                  preferred_element_type=jnp.float32)

    return pl.pallas_call(
        body,
        grid=(NPAD // _BN,),
        in_specs=[
            pl.BlockSpec((_BN, 2), lambda i: (i, 0)),
            pl.BlockSpec((_BN, F_IN), lambda i: (i, 0)),
            pl.BlockSpec((F_IN, fo), lambda i: (0, 0)),
        ],
        out_specs=[
            pl.BlockSpec((_BN, 1), lambda i: (i, 0)),
            pl.BlockSpec((_BN, fo), lambda i: (i, 0)),
        ],
        out_shape=[
            jax.ShapeDtypeStruct((NPAD, 1), jnp.float32),
            jax.ShapeDtypeStruct((NPAD, fo), jnp.float32),
        ],
    )(degT, x_pad, W1)


def _tc_layer(acc, yw, dis, b2d, Wn):
    fi, fo = Wn.shape

    def body(acc_ref, yw_ref, dis_ref, b_ref, w_ref, out_ref):
        s = acc_ref[0] + acc_ref[1] + yw_ref[...]
        h = jnp.maximum(dis_ref[...] * s + b_ref[...], 0.0)
        out_ref[...] = dis_ref[...] * jnp.dot(
            h, w_ref[...], preferred_element_type=jnp.float32)

    return pl.pallas_call(
        body,
        grid=(NPAD // _BN,),
        in_specs=[
            pl.BlockSpec((NC, _BN, fi), lambda i: (0, i, 0)),
            pl.BlockSpec((_BN, fi), lambda i: (i, 0)),
            pl.BlockSpec((_BN, 1), lambda i: (i, 0)),
            pl.BlockSpec((1, fi), lambda i: (0, 0)),
            pl.BlockSpec((fi, fo), lambda i: (0, 0)),
        ],
        out_specs=pl.BlockSpec((_BN, fo), lambda i: (i, 0)),
        out_shape=jax.ShapeDtypeStruct((NPAD, fo), jnp.float32),
    )(acc, yw, dis, b2d, Wn)


def _tc_h3(acc, yw, dis, b2d):
    fi = yw.shape[1]
    bn = 1000

    def body(acc_ref, yw_ref, dis_ref, b_ref, out_ref):
        s = acc_ref[0] + acc_ref[1] + yw_ref[...]
        out_ref[...] = dis_ref[...] * s + b_ref[...]

    return pl.pallas_call(
        body,
        grid=(N // bn,),
        in_specs=[
            pl.BlockSpec((NC, bn, fi), lambda i: (0, i, 0)),
            pl.BlockSpec((bn, fi), lambda i: (i, 0)),
            pl.BlockSpec((bn, 1), lambda i: (i, 0)),
            pl.BlockSpec((1, fi), lambda i: (0, 0)),
        ],
        out_specs=pl.BlockSpec((bn, fi), lambda i: (i, 0)),
        out_shape=jax.ShapeDtypeStruct((N, fi), jnp.float32),
    )(acc, yw, dis, b2d)


def _tc_encoder(h3f, enc1_W, enc1_b, enc2_W, enc2_b, dec1_W, dec1_b):
    K = h3f.shape[1]
    BK = 16000
    steps = K // BK

    def body(h_ref, w_ref, b1_ref, w2_ref, b2_ref, w3_ref, b3_ref,
             out_ref, acc_ref):
        i = pl.program_id(0)
        part = jnp.dot(h_ref[...], w_ref[...],
                       preferred_element_type=jnp.float32)

        @pl.when(i == 0)
        def _():
            acc_ref[...] = part

        @pl.when(i > 0)
        def _():
            acc_ref[...] = acc_ref[...] + part

        @pl.when(i == steps - 1)
        def _():
            z = jnp.maximum(acc_ref[...] + b1_ref[...], 0.0)
            z = jnp.dot(z, w2_ref[...],
                        preferred_element_type=jnp.float32) + b2_ref[...]
            z = jnp.maximum(
                jnp.dot(z, w3_ref[...],
                        preferred_element_type=jnp.float32) + b3_ref[...],
                0.0)
            out_ref[...] = z

    return pl.pallas_call(
        body,
        grid=(steps,),
        in_specs=[
            pl.BlockSpec((1, BK), lambda i: (0, i)),
            pl.BlockSpec((BK, 128), lambda i: (i, 0)),
            pl.BlockSpec((1, 128), lambda i: (0, 0)),
            pl.BlockSpec((128, 64), lambda i: (0, 0)),
            pl.BlockSpec((1, 64), lambda i: (0, 0)),
            pl.BlockSpec((64, 128), lambda i: (0, 0)),
            pl.BlockSpec((1, 128), lambda i: (0, 0)),
        ],
        out_specs=pl.BlockSpec((1, 128), lambda i: (0, 0)),
        out_shape=jax.ShapeDtypeStruct((1, 128), jnp.float32),
        scratch_shapes=[pltpu.VMEM((1, 128), jnp.float32)],
    )(h3f, enc1_W, enc1_b, enc2_W, enc2_b, dec1_W, dec1_b)


def _tc_decoder(z, dec2_W, dec2_b):
    def body(z_ref, w_ref, b_ref, out_ref):
        t = jnp.dot(z_ref[...], w_ref[...],
                    preferred_element_type=jnp.float32) + b_ref[...]
        out_ref[...] = 1.0 / (1.0 + jnp.exp(-t))

    return pl.pallas_call(
        body,
        out_shape=jax.ShapeDtypeStruct((1, N), jnp.float32),
    )(z, dec2_W, dec2_b)


def kernel(x, edge_index, batch_size, batch_index, W1, b1, W2, b2, W3, b3,
           enc1_W, enc1_b, enc2_W, enc2_b, dec1_W, dec1_b, dec2_W, dec2_b):
    del batch_size, batch_index
    pad = jnp.arange(E_PAD - E, dtype=jnp.int32) % (NPAD - N) + N
    src_rows = jnp.concatenate([edge_index[0], pad]).reshape(E_PAD // CHUNK,
                                                             CHUNK)
    dst_rows = jnp.concatenate([edge_index[1], pad]).reshape(E_PAD // CHUNK,
                                                             CHUNK)
    x_pad = jnp.zeros((NPAD, F_IN), jnp.float32).at[:N].set(x)

    zzd = jnp.zeros((NPAD,), jnp.float32)
    zz64 = jnp.zeros((NPAD, 64), jnp.float32)
    zz32 = jnp.zeros((NPAD, 32), jnp.float32)
    zz16 = jnp.zeros((NPAD, 16), jnp.float32)
    ones = jnp.ones((CHUNK,), jnp.float32)

    deg = _sc_degree(dst_rows, zzd, ones)
    degT = deg.T
    dis, yw1 = _tc_prep(degT, x_pad, W1)

    acc1 = _sc_propagate(yw1, src_rows, dst_rows, zz64, 64)
    yw2 = _tc_layer(acc1, yw1, dis, b1.reshape(1, -1), W2)

    acc2 = _sc_propagate(yw2, src_rows, dst_rows, zz32, 32)
    yw3 = _tc_layer(acc2, yw2, dis, b2.reshape(1, -1), W3)

    acc3 = _sc_propagate(yw3, src_rows, dst_rows, zz16, 16)
    h3 = _tc_h3(acc3, yw3, dis, b3.reshape(1, -1))

    h3f = h3.reshape(1, N * 16)
    z = _tc_encoder(h3f, enc1_W, enc1_b.reshape(1, -1), enc2_W,
                    enc2_b.reshape(1, -1), dec1_W, dec1_b.reshape(1, -1))
    return _tc_decoder(z, dec2_W, dec2_b.reshape(1, -1))


# speedup vs baseline: 28.8406x; 1.1273x over previous
import functools

import jax
import jax.numpy as jnp
from jax import lax
from jax.experimental import pallas as pl
from jax.experimental.pallas import tpu as pltpu
from jax.experimental.pallas import tpu_sc as plsc

N = 10000
NPAD = 10240
E = 320000
F_IN = 128
NC = 2
NS = 16
NW = NC * NS
CHUNK = 128
EW = 10240
E_PAD = EW * NW
ROWS_W = EW // CHUNK
NODES_T = NPAD // NS

@functools.cache
def _mesh():
    return plsc.VectorSubcoreMesh(
        core_axis_name="c", subcore_axis_name="s",
        num_cores=NC, num_subcores=NS)


def _sc_degree(dst_rows, zz, ones):
    def body(dst_hbm, zz_hbm, ones_hbm, out_hbm, idx_v, ones_v, zz_v, acc_sh):
        cid = lax.axis_index("c")
        sid = lax.axis_index("s")
        w = sid * NC + cid
        pltpu.sync_copy(zz_hbm.at[pl.ds(sid * NODES_T, NODES_T)], zz_v)
        pltpu.sync_copy(zz_v, acc_sh.at[pl.ds(sid * NODES_T, NODES_T)])
        pltpu.sync_copy(ones_hbm, ones_v)
        plsc.subcore_barrier()
        pltpu.sync_copy(dst_hbm.at[pl.ds(w * ROWS_W, ROWS_W)], idx_v)

        def step(g, carry):
            pltpu.sync_copy(ones_v, acc_sh.at[idx_v.at[g]], add=True)
            return carry

        lax.fori_loop(0, ROWS_W, step, 0)
        plsc.subcore_barrier()
        pltpu.sync_copy(acc_sh.at[pl.ds(sid * NODES_T, NODES_T)],
                        out_hbm.at[cid, pl.ds(sid * NODES_T, NODES_T)])

    k = pl.kernel(
        body,
        out_type=jax.ShapeDtypeStruct((NC, NPAD), jnp.float32),
        mesh=_mesh(),
        compiler_params=pltpu.CompilerParams(use_tc_tiling_on_sc=False),
        scratch_types=[
            pltpu.VMEM((ROWS_W, CHUNK), jnp.int32),
            pltpu.VMEM((CHUNK,), jnp.float32),
            pltpu.VMEM((NODES_T,), jnp.float32),
            pltpu.VMEM_SHARED((NPAD,), jnp.float32),
        ],
    )
    return k(dst_rows, zz, ones)


def _sc_propagate(yw, src_rows, dst_rows, zz, F):
    def body(yw_hbm, src_hbm, dst_hbm, zz_hbm, out_hbm,
             sidx_v, didx_v, rows_a, rows_b, zz_v, acc_sh, sem_a, sem_b):
        cid = lax.axis_index("c")
        sid = lax.axis_index("s")
        w = sid * NC + cid
        pltpu.sync_copy(zz_hbm.at[pl.ds(sid * NODES_T, NODES_T)], zz_v)
        pltpu.sync_copy(zz_v, acc_sh.at[pl.ds(sid * NODES_T, NODES_T)])
        plsc.subcore_barrier()
        pltpu.sync_copy(src_hbm.at[pl.ds(w * ROWS_W, ROWS_W)], sidx_v)
        pltpu.sync_copy(dst_hbm.at[pl.ds(w * ROWS_W, ROWS_W)], didx_v)

        def gstart(g, buf, sem):
            pltpu.async_copy(yw_hbm.at[sidx_v.at[g]], buf, sem)

        def gwait(buf, sem):
            pltpu.make_async_copy(yw_hbm.at[sidx_v.at[0]], buf, sem).wait()

        gstart(0, rows_a, sem_a)

        def pair(p, carry):
            t0 = 2 * p
            t1 = t0 + 1
            gwait(rows_a, sem_a)
            gstart(t1, rows_b, sem_b)
            pltpu.sync_copy(rows_a, acc_sh.at[didx_v.at[t0]], add=True)
            gwait(rows_b, sem_b)
            gstart(jnp.minimum(t1 + 1, ROWS_W - 1), rows_a, sem_a)
            pltpu.sync_copy(rows_b, acc_sh.at[didx_v.at[t1]], add=True)
            return carry

        lax.fori_loop(0, ROWS_W // 2, pair, 0)
        gwait(rows_a, sem_a)
        plsc.subcore_barrier()
        pltpu.sync_copy(acc_sh.at[pl.ds(sid * NODES_T, NODES_T)],
                        out_hbm.at[cid, pl.ds(sid * NODES_T, NODES_T)])

    k = pl.kernel(
        body,
        out_type=jax.ShapeDtypeStruct((NC, NPAD, F), jnp.float32),
        mesh=_mesh(),
        compiler_params=pltpu.CompilerParams(use_tc_tiling_on_sc=False),
        scratch_types=[
            pltpu.VMEM((ROWS_W, CHUNK), jnp.int32),
            pltpu.VMEM((ROWS_W, CHUNK), jnp.int32),
            pltpu.VMEM((CHUNK, F), jnp.float32),
            pltpu.VMEM((CHUNK, F), jnp.float32),
            pltpu.VMEM((NODES_T, F), jnp.float32),
            pltpu.VMEM_SHARED((NPAD, F), jnp.float32),
            pltpu.SemaphoreType.DMA,
            pltpu.SemaphoreType.DMA,
        ],
    )
    return k(yw, src_rows, dst_rows, zz)


_BN = 1024


def _tc_prep(degT, x_pad, W1):
    fo = W1.shape[1]

    def body(deg_ref, x_ref, w_ref, dis_ref, yw_ref):
        d = lax.rsqrt(deg_ref[:, 0:1] + deg_ref[:, 1:2] + 1.0)
        dis_ref[...] = d
        yw_ref[...] = d * jnp.dot(x_ref[...], w_ref[...],
                                  preferred_element_type=jnp.float32)

    return pl.pallas_call(
        body,
        grid=(NPAD // _BN,),
        in_specs=[
            pl.BlockSpec((_BN, 2), lambda i: (i, 0)),
            pl.BlockSpec((_BN, F_IN), lambda i: (i, 0)),
            pl.BlockSpec((F_IN, fo), lambda i: (0, 0)),
        ],
        out_specs=[
            pl.BlockSpec((_BN, 1), lambda i: (i, 0)),
            pl.BlockSpec((_BN, fo), lambda i: (i, 0)),
        ],
        out_shape=[
            jax.ShapeDtypeStruct((NPAD, 1), jnp.float32),
            jax.ShapeDtypeStruct((NPAD, fo), jnp.float32),
        ],
    )(degT, x_pad, W1)


def _tc_layer(acc, yw, dis, b2d, Wn):
    fi, fo = Wn.shape

    def body(acc_ref, yw_ref, dis_ref, b_ref, w_ref, out_ref):
        s = acc_ref[0] + acc_ref[1] + yw_ref[...]
        h = jnp.maximum(dis_ref[...] * s + b_ref[...], 0.0)
        out_ref[...] = dis_ref[...] * jnp.dot(
            h, w_ref[...], preferred_element_type=jnp.float32)

    return pl.pallas_call(
        body,
        grid=(NPAD // _BN,),
        in_specs=[
            pl.BlockSpec((NC, _BN, fi), lambda i: (0, i, 0)),
            pl.BlockSpec((_BN, fi), lambda i: (i, 0)),
            pl.BlockSpec((_BN, 1), lambda i: (i, 0)),
            pl.BlockSpec((1, fi), lambda i: (0, 0)),
            pl.BlockSpec((fi, fo), lambda i: (0, 0)),
        ],
        out_specs=pl.BlockSpec((_BN, fo), lambda i: (i, 0)),
        out_shape=jax.ShapeDtypeStruct((NPAD, fo), jnp.float32),
    )(acc, yw, dis, b2d, Wn)


def _tc_h3(acc, yw, dis, b2d):
    fi = yw.shape[1]
    bn = 1000

    def body(acc_ref, yw_ref, dis_ref, b_ref, out_ref):
        s = acc_ref[0] + acc_ref[1] + yw_ref[...]
        out_ref[...] = dis_ref[...] * s + b_ref[...]

    return pl.pallas_call(
        body,
        grid=(N // bn,),
        in_specs=[
            pl.BlockSpec((NC, bn, fi), lambda i: (0, i, 0)),
            pl.BlockSpec((bn, fi), lambda i: (i, 0)),
            pl.BlockSpec((bn, 1), lambda i: (i, 0)),
            pl.BlockSpec((1, fi), lambda i: (0, 0)),
        ],
        out_specs=pl.BlockSpec((bn, fi), lambda i: (i, 0)),
        out_shape=jax.ShapeDtypeStruct((N, fi), jnp.float32),
    )(acc, yw, dis, b2d)


def _tc_encoder(h3f, enc1_W, enc1_b, enc2_W, enc2_b, dec1_W, dec1_b):
    K = h3f.shape[1]
    BK = 16000
    steps = K // BK

    def body(h_ref, w_ref, b1_ref, w2_ref, b2_ref, w3_ref, b3_ref,
             out_ref, acc_ref):
        i = pl.program_id(0)
        part = jnp.dot(h_ref[...], w_ref[...],
                       preferred_element_type=jnp.float32)

        @pl.when(i == 0)
        def _():
            acc_ref[...] = part

        @pl.when(i > 0)
        def _():
            acc_ref[...] = acc_ref[...] + part

        @pl.when(i == steps - 1)
        def _():
            z = jnp.maximum(acc_ref[...] + b1_ref[...], 0.0)
            z = jnp.dot(z, w2_ref[...],
                        preferred_element_type=jnp.float32) + b2_ref[...]
            z = jnp.maximum(
                jnp.dot(z, w3_ref[...],
                        preferred_element_type=jnp.float32) + b3_ref[...],
                0.0)
            out_ref[...] = z

    return pl.pallas_call(
        body,
        grid=(steps,),
        in_specs=[
            pl.BlockSpec((1, BK), lambda i: (0, i)),
            pl.BlockSpec((BK, 128), lambda i: (i, 0)),
            pl.BlockSpec((1, 128), lambda i: (0, 0)),
            pl.BlockSpec((128, 64), lambda i: (0, 0)),
            pl.BlockSpec((1, 64), lambda i: (0, 0)),
            pl.BlockSpec((64, 128), lambda i: (0, 0)),
            pl.BlockSpec((1, 128), lambda i: (0, 0)),
        ],
        out_specs=pl.BlockSpec((1, 128), lambda i: (0, 0)),
        out_shape=jax.ShapeDtypeStruct((1, 128), jnp.float32),
        scratch_shapes=[pltpu.VMEM((1, 128), jnp.float32)],
    )(h3f, enc1_W, enc1_b, enc2_W, enc2_b, dec1_W, dec1_b)


def _tc_decoder(z, dec2_W, dec2_b):
    def body(z_ref, w_ref, b_ref, out_ref):
        t = jnp.dot(z_ref[...], w_ref[...],
                    preferred_element_type=jnp.float32) + b_ref[...]
        out_ref[...] = 1.0 / (1.0 + jnp.exp(-t))

    return pl.pallas_call(
        body,
        out_shape=jax.ShapeDtypeStruct((1, N), jnp.float32),
    )(z, dec2_W, dec2_b)


def kernel(x, edge_index, batch_size, batch_index, W1, b1, W2, b2, W3, b3,
           enc1_W, enc1_b, enc2_W, enc2_b, dec1_W, dec1_b, dec2_W, dec2_b):
    del batch_size, batch_index
    pad = jnp.arange(E_PAD - E, dtype=jnp.int32) % (NPAD - N) + N
    src_rows = jnp.concatenate([edge_index[0], pad]).reshape(E_PAD // CHUNK,
                                                             CHUNK)
    dst_rows = jnp.concatenate([edge_index[1], pad]).reshape(E_PAD // CHUNK,
                                                             CHUNK)
    x_pad = jnp.zeros((NPAD, F_IN), jnp.float32).at[:N].set(x)

    zzd = jnp.zeros((NPAD,), jnp.float32)
    zz64 = jnp.zeros((NPAD, 64), jnp.float32)
    zz32 = jnp.zeros((NPAD, 32), jnp.float32)
    zz16 = jnp.zeros((NPAD, 16), jnp.float32)
    ones = jnp.ones((CHUNK,), jnp.float32)

    deg = _sc_degree(dst_rows, zzd, ones)
    degT = deg.T
    dis, yw1 = _tc_prep(degT, x_pad, W1)

    acc1 = _sc_propagate(yw1, src_rows, dst_rows, zz64, 64)
    yw2 = _tc_layer(acc1, yw1, dis, b1.reshape(1, -1), W2)

    acc2 = _sc_propagate(yw2, src_rows, dst_rows, zz32, 32)
    yw3 = _tc_layer(acc2, yw2, dis, b2.reshape(1, -1), W3)

    acc3 = _sc_propagate(yw3, src_rows, dst_rows, zz16, 16)
    h3 = _tc_h3(acc3, yw3, dis, b3.reshape(1, -1))

    h3f = h3.reshape(1, N * 16)
    z = _tc_encoder(h3f, enc1_W, enc1_b.reshape(1, -1), enc2_W,
                    enc2_b.reshape(1, -1), dec1_W, dec1_b.reshape(1, -1))
    return _tc_decoder(z, dec2_W, dec2_b.reshape(1, -1))


# speedup vs baseline: 38.0546x; 1.3195x over previous
import functools

import jax
import jax.numpy as jnp
from jax import lax
from jax.experimental import pallas as pl
from jax.experimental.pallas import tpu as pltpu
from jax.experimental.pallas import tpu_sc as plsc

N = 10000
NPAD = 10240
E = 320000
F_IN = 128
NC = 2
NS = 16
NW = NC * NS
CHUNK = 128
EW = 10240
E_PAD = EW * NW
ROWS_W = EW // CHUNK
NODES_T = NPAD // NS

@functools.cache
def _mesh():
    return plsc.VectorSubcoreMesh(
        core_axis_name="c", subcore_axis_name="s",
        num_cores=NC, num_subcores=NS)


def _sc_degree(dst_rows, zz, ones):
    K = 4

    def body(dst_hbm, zz_hbm, ones_hbm, out_hbm, idx_v, ones_v, acc_sh, sem):
        cid = lax.axis_index("c")
        sid = lax.axis_index("s")
        w = sid * NC + cid
        pltpu.sync_copy(zz_hbm.at[pl.ds(sid * NODES_T, NODES_T)],
                        acc_sh.at[pl.ds(sid * NODES_T, NODES_T)])
        pltpu.sync_copy(ones_hbm, ones_v)
        plsc.subcore_barrier()
        pltpu.sync_copy(dst_hbm.at[pl.ds(w * ROWS_W, ROWS_W)], idx_v)

        def step(s, carry):
            for j in range(K):
                pltpu.async_copy(ones_v, acc_sh.at[idx_v.at[s * K + j]], sem,
                                 add=True)
            for j in range(K):
                pltpu.make_async_copy(ones_v, acc_sh.at[idx_v.at[0]],
                                      sem).wait()
            return carry

        lax.fori_loop(0, ROWS_W // K, step, 0)
        plsc.subcore_barrier()
        pltpu.sync_copy(acc_sh.at[pl.ds(sid * NODES_T, NODES_T)],
                        out_hbm.at[cid, pl.ds(sid * NODES_T, NODES_T)])

    k = pl.kernel(
        body,
        out_type=jax.ShapeDtypeStruct((NC, NPAD), jnp.float32),
        mesh=_mesh(),
        compiler_params=pltpu.CompilerParams(use_tc_tiling_on_sc=False),
        scratch_types=[
            pltpu.VMEM((ROWS_W, CHUNK), jnp.int32),
            pltpu.VMEM((CHUNK,), jnp.float32),
            pltpu.VMEM_SHARED((NPAD,), jnp.float32),
            pltpu.SemaphoreType.DMA,
        ],
    )
    return k(dst_rows, zz, ones)


def _sc_propagate(yw, src_rows, dst_rows, zz, F):
    K = 4
    NSUP = ROWS_W // K

    def body(yw_hbm, src_hbm, dst_hbm, zz_hbm, out_hbm,
             sidx_v, didx_v, buf_a, buf_b, acc_sh, sem_ga, sem_gb, sem_s):
        cid = lax.axis_index("c")
        sid = lax.axis_index("s")
        w = sid * NC + cid
        pltpu.sync_copy(zz_hbm.at[pl.ds(sid * NODES_T, NODES_T)],
                        acc_sh.at[pl.ds(sid * NODES_T, NODES_T)])
        plsc.subcore_barrier()
        pltpu.sync_copy(src_hbm.at[pl.ds(w * ROWS_W, ROWS_W)], sidx_v)
        pltpu.sync_copy(dst_hbm.at[pl.ds(w * ROWS_W, ROWS_W)], didx_v)

        def fire(s, buf, sem):
            for j in range(K):
                pltpu.async_copy(yw_hbm.at[sidx_v.at[s * K + j]],
                                 buf.at[pl.ds(j * CHUNK, CHUNK)], sem)

        def drain(buf, sem):
            for j in range(K):
                pltpu.make_async_copy(yw_hbm.at[sidx_v.at[0]],
                                      buf.at[pl.ds(j * CHUNK, CHUNK)],
                                      sem).wait()

        def scat(s, buf):
            for j in range(K):
                pltpu.async_copy(buf.at[pl.ds(j * CHUNK, CHUNK)],
                                 acc_sh.at[didx_v.at[s * K + j]], sem_s,
                                 add=True)
            for j in range(K):
                pltpu.make_async_copy(buf.at[pl.ds(j * CHUNK, CHUNK)],
                                      acc_sh.at[didx_v.at[0]], sem_s).wait()

        fire(0, buf_a, sem_ga)

        def pair(p, carry):
            s0 = 2 * p
            s1 = s0 + 1
            drain(buf_a, sem_ga)
            fire(s1, buf_b, sem_gb)
            scat(s0, buf_a)
            drain(buf_b, sem_gb)
            fire(jnp.minimum(s1 + 1, NSUP - 1), buf_a, sem_ga)
            scat(s1, buf_b)
            return carry

        lax.fori_loop(0, NSUP // 2, pair, 0)
        drain(buf_a, sem_ga)
        plsc.subcore_barrier()
        pltpu.sync_copy(acc_sh.at[pl.ds(sid * NODES_T, NODES_T)],
                        out_hbm.at[cid, pl.ds(sid * NODES_T, NODES_T)])

    k = pl.kernel(
        body,
        out_type=jax.ShapeDtypeStruct((NC, NPAD, F), jnp.float32),
        mesh=_mesh(),
        compiler_params=pltpu.CompilerParams(use_tc_tiling_on_sc=False),
        scratch_types=[
            pltpu.VMEM((ROWS_W, CHUNK), jnp.int32),
            pltpu.VMEM((ROWS_W, CHUNK), jnp.int32),
            pltpu.VMEM((K * CHUNK, F), jnp.float32),
            pltpu.VMEM((K * CHUNK, F), jnp.float32),
            pltpu.VMEM_SHARED((NPAD, F), jnp.float32),
            pltpu.SemaphoreType.DMA,
            pltpu.SemaphoreType.DMA,
            pltpu.SemaphoreType.DMA,
        ],
    )
    return k(yw, src_rows, dst_rows, zz)


_BN = 1024


def _tc_prep(degT, x_pad, W1):
    fo = W1.shape[1]

    def body(deg_ref, x_ref, w_ref, dis_ref, yw_ref):
        d = lax.rsqrt(deg_ref[:, 0:1] + deg_ref[:, 1:2] + 1.0)
        dis_ref[...] = d
        yw_ref[...] = d * jnp.dot(x_ref[...], w_ref[...],
                                  preferred_element_type=jnp.float32)

    return pl.pallas_call(
        body,
        grid=(NPAD // _BN,),
        in_specs=[
            pl.BlockSpec((_BN, 2), lambda i: (i, 0)),
            pl.BlockSpec((_BN, F_IN), lambda i: (i, 0)),
            pl.BlockSpec((F_IN, fo), lambda i: (0, 0)),
        ],
        out_specs=[
            pl.BlockSpec((_BN, 1), lambda i: (i, 0)),
            pl.BlockSpec((_BN, fo), lambda i: (i, 0)),
        ],
        out_shape=[
            jax.ShapeDtypeStruct((NPAD, 1), jnp.float32),
            jax.ShapeDtypeStruct((NPAD, fo), jnp.float32),
        ],
    )(degT, x_pad, W1)


def _tc_layer(acc, yw, dis, b2d, Wn):
    fi, fo = Wn.shape

    def body(acc_ref, yw_ref, dis_ref, b_ref, w_ref, out_ref):
        s = acc_ref[0] + acc_ref[1] + yw_ref[...]
        h = jnp.maximum(dis_ref[...] * s + b_ref[...], 0.0)
        out_ref[...] = dis_ref[...] * jnp.dot(
            h, w_ref[...], preferred_element_type=jnp.float32)

    return pl.pallas_call(
        body,
        grid=(NPAD // _BN,),
        in_specs=[
            pl.BlockSpec((NC, _BN, fi), lambda i: (0, i, 0)),
            pl.BlockSpec((_BN, fi), lambda i: (i, 0)),
            pl.BlockSpec((_BN, 1), lambda i: (i, 0)),
            pl.BlockSpec((1, fi), lambda i: (0, 0)),
            pl.BlockSpec((fi, fo), lambda i: (0, 0)),
        ],
        out_specs=pl.BlockSpec((_BN, fo), lambda i: (i, 0)),
        out_shape=jax.ShapeDtypeStruct((NPAD, fo), jnp.float32),
    )(acc, yw, dis, b2d, Wn)


def _tc_h3(acc, yw, dis, b2d):
    fi = yw.shape[1]
    bn = 1000

    def body(acc_ref, yw_ref, dis_ref, b_ref, out_ref):
        s = acc_ref[0] + acc_ref[1] + yw_ref[...]
        out_ref[...] = dis_ref[...] * s + b_ref[...]

    return pl.pallas_call(
        body,
        grid=(N // bn,),
        in_specs=[
            pl.BlockSpec((NC, bn, fi), lambda i: (0, i, 0)),
            pl.BlockSpec((bn, fi), lambda i: (i, 0)),
            pl.BlockSpec((bn, 1), lambda i: (i, 0)),
            pl.BlockSpec((1, fi), lambda i: (0, 0)),
        ],
        out_specs=pl.BlockSpec((bn, fi), lambda i: (i, 0)),
        out_shape=jax.ShapeDtypeStruct((N, fi), jnp.float32),
    )(acc, yw, dis, b2d)


def _tc_encoder(h3f, enc1_W, enc1_b, enc2_W, enc2_b, dec1_W, dec1_b):
    K = h3f.shape[1]
    BK = 16000
    steps = K // BK

    def body(h_ref, w_ref, b1_ref, w2_ref, b2_ref, w3_ref, b3_ref,
             out_ref, acc_ref):
        i = pl.program_id(0)
        part = jnp.dot(h_ref[...], w_ref[...],
                       preferred_element_type=jnp.float32)

        @pl.when(i == 0)
        def _():
            acc_ref[...] = part

        @pl.when(i > 0)
        def _():
            acc_ref[...] = acc_ref[...] + part

        @pl.when(i == steps - 1)
        def _():
            z = jnp.maximum(acc_ref[...] + b1_ref[...], 0.0)
            z = jnp.dot(z, w2_ref[...],
                        preferred_element_type=jnp.float32) + b2_ref[...]
            z = jnp.maximum(
                jnp.dot(z, w3_ref[...],
                        preferred_element_type=jnp.float32) + b3_ref[...],
                0.0)
            out_ref[...] = z

    return pl.pallas_call(
        body,
        grid=(steps,),
        in_specs=[
            pl.BlockSpec((1, BK), lambda i: (0, i)),
            pl.BlockSpec((BK, 128), lambda i: (i, 0)),
            pl.BlockSpec((1, 128), lambda i: (0, 0)),
            pl.BlockSpec((128, 64), lambda i: (0, 0)),
            pl.BlockSpec((1, 64), lambda i: (0, 0)),
            pl.BlockSpec((64, 128), lambda i: (0, 0)),
            pl.BlockSpec((1, 128), lambda i: (0, 0)),
        ],
        out_specs=pl.BlockSpec((1, 128), lambda i: (0, 0)),
        out_shape=jax.ShapeDtypeStruct((1, 128), jnp.float32),
        scratch_shapes=[pltpu.VMEM((1, 128), jnp.float32)],
    )(h3f, enc1_W, enc1_b, enc2_W, enc2_b, dec1_W, dec1_b)


def _tc_decoder(z, dec2_W, dec2_b):
    def body(z_ref, w_ref, b_ref, out_ref):
        t = jnp.dot(z_ref[...], w_ref[...],
                    preferred_element_type=jnp.float32) + b_ref[...]
        out_ref[...] = 1.0 / (1.0 + jnp.exp(-t))

    return pl.pallas_call(
        body,
        out_shape=jax.ShapeDtypeStruct((1, N), jnp.float32),
    )(z, dec2_W, dec2_b)


def kernel(x, edge_index, batch_size, batch_index, W1, b1, W2, b2, W3, b3,
           enc1_W, enc1_b, enc2_W, enc2_b, dec1_W, dec1_b, dec2_W, dec2_b):
    del batch_size, batch_index
    pad = jnp.arange(E_PAD - E, dtype=jnp.int32) % (NPAD - N) + N
    src_rows = jnp.concatenate([edge_index[0], pad]).reshape(E_PAD // CHUNK,
                                                             CHUNK)
    dst_rows = jnp.concatenate([edge_index[1], pad]).reshape(E_PAD // CHUNK,
                                                             CHUNK)
    x_pad = jnp.zeros((NPAD, F_IN), jnp.float32).at[:N].set(x)

    zzd = jnp.zeros((NPAD,), jnp.float32)
    zz64 = jnp.zeros((NPAD, 64), jnp.float32)
    zz32 = jnp.zeros((NPAD, 32), jnp.float32)
    zz16 = jnp.zeros((NPAD, 16), jnp.float32)
    ones = jnp.ones((CHUNK,), jnp.float32)

    deg = _sc_degree(dst_rows, zzd, ones)
    degT = deg.T
    dis, yw1 = _tc_prep(degT, x_pad, W1)

    acc1 = _sc_propagate(yw1, src_rows, dst_rows, zz64, 64)
    yw2 = _tc_layer(acc1, yw1, dis, b1.reshape(1, -1), W2)

    acc2 = _sc_propagate(yw2, src_rows, dst_rows, zz32, 32)
    yw3 = _tc_layer(acc2, yw2, dis, b2.reshape(1, -1), W3)

    acc3 = _sc_propagate(yw3, src_rows, dst_rows, zz16, 16)
    h3 = _tc_h3(acc3, yw3, dis, b3.reshape(1, -1))

    h3f = h3.reshape(1, N * 16)
    z = _tc_encoder(h3f, enc1_W, enc1_b.reshape(1, -1), enc2_W,
                    enc2_b.reshape(1, -1), dec1_W, dec1_b.reshape(1, -1))
    return _tc_decoder(z, dec2_W, dec2_b.reshape(1, -1))


# speedup vs baseline: 39.4920x; 1.0378x over previous
import functools

import jax
import jax.numpy as jnp
from jax import lax
from jax.experimental import pallas as pl
from jax.experimental.pallas import tpu as pltpu
from jax.experimental.pallas import tpu_sc as plsc

N = 10000
NPAD = 10240
E = 320000
F_IN = 128
NC = 2
NS = 16
NW = NC * NS
CHUNK = 128
EW = 10240
E_PAD = EW * NW
ROWS_W = EW // CHUNK
NODES_T = NPAD // NS

@functools.cache
def _mesh():
    return plsc.VectorSubcoreMesh(
        core_axis_name="c", subcore_axis_name="s",
        num_cores=NC, num_subcores=NS)


def _sc_degree(dst_rows, zz, ones):
    K = 4

    def body(dst_hbm, zz_hbm, ones_hbm, out_hbm, idx_v, ones_v, acc_sh, sem):
        cid = lax.axis_index("c")
        sid = lax.axis_index("s")
        w = sid * NC + cid
        pltpu.sync_copy(zz_hbm.at[pl.ds(sid * NODES_T, NODES_T)],
                        acc_sh.at[pl.ds(sid * NODES_T, NODES_T)])
        pltpu.sync_copy(ones_hbm, ones_v)
        plsc.subcore_barrier()
        pltpu.sync_copy(dst_hbm.at[pl.ds(w * ROWS_W, ROWS_W)], idx_v)

        def step(s, carry):
            for j in range(K):
                pltpu.async_copy(ones_v, acc_sh.at[idx_v.at[s * K + j]], sem,
                                 add=True)
            for j in range(K):
                pltpu.make_async_copy(ones_v, acc_sh.at[idx_v.at[0]],
                                      sem).wait()
            return carry

        lax.fori_loop(0, ROWS_W // K, step, 0)
        plsc.subcore_barrier()
        pltpu.sync_copy(acc_sh.at[pl.ds(sid * NODES_T, NODES_T)],
                        out_hbm.at[cid, pl.ds(sid * NODES_T, NODES_T)])

    k = pl.kernel(
        body,
        out_type=jax.ShapeDtypeStruct((NC, NPAD), jnp.float32),
        mesh=_mesh(),
        compiler_params=pltpu.CompilerParams(use_tc_tiling_on_sc=False),
        scratch_types=[
            pltpu.VMEM((ROWS_W, CHUNK), jnp.int32),
            pltpu.VMEM((CHUNK,), jnp.float32),
            pltpu.VMEM_SHARED((NPAD,), jnp.float32),
            pltpu.SemaphoreType.DMA,
        ],
    )
    return k(dst_rows, zz, ones)


def _sc_propagate(yw, src_rows, dst_rows, zz, F):
    K = 4 if F >= 64 else 8
    NSUP = ROWS_W // K

    def body(yw_hbm, src_hbm, dst_hbm, zz_hbm, out_hbm,
             sidx_v, didx_v, buf_a, buf_b, acc_sh, sem_ga, sem_gb, sem_s):
        cid = lax.axis_index("c")
        sid = lax.axis_index("s")
        w = sid * NC + cid
        pltpu.sync_copy(zz_hbm.at[pl.ds(sid * NODES_T, NODES_T)],
                        acc_sh.at[pl.ds(sid * NODES_T, NODES_T)])
        plsc.subcore_barrier()
        pltpu.sync_copy(src_hbm.at[pl.ds(w * ROWS_W, ROWS_W)], sidx_v)
        pltpu.sync_copy(dst_hbm.at[pl.ds(w * ROWS_W, ROWS_W)], didx_v)

        def fire(s, buf, sem):
            for j in range(K):
                pltpu.async_copy(yw_hbm.at[sidx_v.at[s * K + j]],
                                 buf.at[pl.ds(j * CHUNK, CHUNK)], sem)

        def drain(buf, sem):
            for j in range(K):
                pltpu.make_async_copy(yw_hbm.at[sidx_v.at[0]],
                                      buf.at[pl.ds(j * CHUNK, CHUNK)],
                                      sem).wait()

        def scat(s, buf):
            for j in range(K):
                pltpu.async_copy(buf.at[pl.ds(j * CHUNK, CHUNK)],
                                 acc_sh.at[didx_v.at[s * K + j]], sem_s,
                                 add=True)
            for j in range(K):
                pltpu.make_async_copy(buf.at[pl.ds(j * CHUNK, CHUNK)],
                                      acc_sh.at[didx_v.at[0]], sem_s).wait()

        fire(0, buf_a, sem_ga)

        def pair(p, carry):
            s0 = 2 * p
            s1 = s0 + 1
            drain(buf_a, sem_ga)
            fire(s1, buf_b, sem_gb)
            scat(s0, buf_a)
            drain(buf_b, sem_gb)
            fire(jnp.minimum(s1 + 1, NSUP - 1), buf_a, sem_ga)
            scat(s1, buf_b)
            return carry

        lax.fori_loop(0, NSUP // 2, pair, 0)
        drain(buf_a, sem_ga)
        plsc.subcore_barrier()
        pltpu.sync_copy(acc_sh.at[pl.ds(sid * NODES_T, NODES_T)],
                        out_hbm.at[cid, pl.ds(sid * NODES_T, NODES_T)])

    k = pl.kernel(
        body,
        out_type=jax.ShapeDtypeStruct((NC, NPAD, F), jnp.float32),
        mesh=_mesh(),
        compiler_params=pltpu.CompilerParams(use_tc_tiling_on_sc=False),
        scratch_types=[
            pltpu.VMEM((ROWS_W, CHUNK), jnp.int32),
            pltpu.VMEM((ROWS_W, CHUNK), jnp.int32),
            pltpu.VMEM((K * CHUNK, F), jnp.float32),
            pltpu.VMEM((K * CHUNK, F), jnp.float32),
            pltpu.VMEM_SHARED((NPAD, F), jnp.float32),
            pltpu.SemaphoreType.DMA,
            pltpu.SemaphoreType.DMA,
            pltpu.SemaphoreType.DMA,
        ],
    )
    return k(yw, src_rows, dst_rows, zz)


_BN = 1000


def _tc_prep(degT, x, W1):
    fo = W1.shape[1]

    def body(deg_ref, x_ref, w_ref, dis_ref, yw_ref):
        d = lax.rsqrt(deg_ref[:, 0:1] + deg_ref[:, 1:2] + 1.0)
        dis_ref[...] = d
        yw_ref[...] = d * jnp.dot(x_ref[...], w_ref[...],
                                  preferred_element_type=jnp.float32)

    return pl.pallas_call(
        body,
        grid=(N // _BN,),
        in_specs=[
            pl.BlockSpec((_BN, 2), lambda i: (i, 0)),
            pl.BlockSpec((_BN, F_IN), lambda i: (i, 0)),
            pl.BlockSpec((F_IN, fo), lambda i: (0, 0)),
        ],
        out_specs=[
            pl.BlockSpec((_BN, 1), lambda i: (i, 0)),
            pl.BlockSpec((_BN, fo), lambda i: (i, 0)),
        ],
        out_shape=[
            jax.ShapeDtypeStruct((N, 1), jnp.float32),
            jax.ShapeDtypeStruct((N, fo), jnp.float32),
        ],
    )(degT, x, W1)


def _tc_layer(acc, yw, dis, b2d, Wn):
    fi, fo = Wn.shape

    def body(acc_ref, yw_ref, dis_ref, b_ref, w_ref, out_ref):
        s = acc_ref[0] + acc_ref[1] + yw_ref[...]
        h = jnp.maximum(dis_ref[...] * s + b_ref[...], 0.0)
        out_ref[...] = dis_ref[...] * jnp.dot(
            h, w_ref[...], preferred_element_type=jnp.float32)

    return pl.pallas_call(
        body,
        grid=(N // _BN,),
        in_specs=[
            pl.BlockSpec((NC, _BN, fi), lambda i: (0, i, 0)),
            pl.BlockSpec((_BN, fi), lambda i: (i, 0)),
            pl.BlockSpec((_BN, 1), lambda i: (i, 0)),
            pl.BlockSpec((1, fi), lambda i: (0, 0)),
            pl.BlockSpec((fi, fo), lambda i: (0, 0)),
        ],
        out_specs=pl.BlockSpec((_BN, fo), lambda i: (i, 0)),
        out_shape=jax.ShapeDtypeStruct((N, fo), jnp.float32),
    )(acc, yw, dis, b2d, Wn)


def _tc_h3(acc, yw, dis, b2d):
    fi = yw.shape[1]
    bn = 1000

    def body(acc_ref, yw_ref, dis_ref, b_ref, out_ref):
        s = acc_ref[0] + acc_ref[1] + yw_ref[...]
        out_ref[...] = dis_ref[...] * s + b_ref[...]

    return pl.pallas_call(
        body,
        grid=(N // bn,),
        in_specs=[
            pl.BlockSpec((NC, bn, fi), lambda i: (0, i, 0)),
            pl.BlockSpec((bn, fi), lambda i: (i, 0)),
            pl.BlockSpec((bn, 1), lambda i: (i, 0)),
            pl.BlockSpec((1, fi), lambda i: (0, 0)),
        ],
        out_specs=pl.BlockSpec((bn, fi), lambda i: (i, 0)),
        out_shape=jax.ShapeDtypeStruct((N, fi), jnp.float32),
    )(acc, yw, dis, b2d)


def _tc_encoder(h3f, enc1_W, enc1_b, enc2_W, enc2_b, dec1_W, dec1_b):
    K = h3f.shape[1]
    BK = 16000
    steps = K // BK

    def body(h_ref, w_ref, b1_ref, w2_ref, b2_ref, w3_ref, b3_ref,
             out_ref, acc_ref):
        i = pl.program_id(0)
        part = jnp.dot(h_ref[...], w_ref[...],
                       preferred_element_type=jnp.float32)

        @pl.when(i == 0)
        def _():
            acc_ref[...] = part

        @pl.when(i > 0)
        def _():
            acc_ref[...] = acc_ref[...] + part

        @pl.when(i == steps - 1)
        def _():
            z = jnp.maximum(acc_ref[...] + b1_ref[...], 0.0)
            z = jnp.dot(z, w2_ref[...],
                        preferred_element_type=jnp.float32) + b2_ref[...]
            z = jnp.maximum(
                jnp.dot(z, w3_ref[...],
                        preferred_element_type=jnp.float32) + b3_ref[...],
                0.0)
            out_ref[...] = z

    return pl.pallas_call(
        body,
        grid=(steps,),
        in_specs=[
            pl.BlockSpec((1, BK), lambda i: (0, i)),
            pl.BlockSpec((BK, 128), lambda i: (i, 0)),
            pl.BlockSpec((1, 128), lambda i: (0, 0)),
            pl.BlockSpec((128, 64), lambda i: (0, 0)),
            pl.BlockSpec((1, 64), lambda i: (0, 0)),
            pl.BlockSpec((64, 128), lambda i: (0, 0)),
            pl.BlockSpec((1, 128), lambda i: (0, 0)),
        ],
        out_specs=pl.BlockSpec((1, 128), lambda i: (0, 0)),
        out_shape=jax.ShapeDtypeStruct((1, 128), jnp.float32),
        scratch_shapes=[pltpu.VMEM((1, 128), jnp.float32)],
    )(h3f, enc1_W, enc1_b, enc2_W, enc2_b, dec1_W, dec1_b)


def _tc_decoder(z, dec2_W, dec2_b):
    def body(z_ref, w_ref, b_ref, out_ref):
        t = jnp.dot(z_ref[...], w_ref[...],
                    preferred_element_type=jnp.float32) + b_ref[...]
        out_ref[...] = 1.0 / (1.0 + jnp.exp(-t))

    return pl.pallas_call(
        body,
        out_shape=jax.ShapeDtypeStruct((1, N), jnp.float32),
    )(z, dec2_W, dec2_b)


def kernel(x, edge_index, batch_size, batch_index, W1, b1, W2, b2, W3, b3,
           enc1_W, enc1_b, enc2_W, enc2_b, dec1_W, dec1_b, dec2_W, dec2_b):
    del batch_size, batch_index
    pad = jnp.arange(E_PAD - E, dtype=jnp.int32) % (NPAD - N)
    src_rows = jnp.concatenate([edge_index[0], pad]).reshape(E_PAD // CHUNK,
                                                             CHUNK)
    dst_rows = jnp.concatenate([edge_index[1], pad + N]).reshape(
        E_PAD // CHUNK, CHUNK)

    zzd = jnp.zeros((NPAD,), jnp.float32)
    zz64 = jnp.zeros((NPAD, 64), jnp.float32)
    zz32 = jnp.zeros((NPAD, 32), jnp.float32)
    zz16 = jnp.zeros((NPAD, 16), jnp.float32)
    ones = jnp.ones((CHUNK,), jnp.float32)

    deg = _sc_degree(dst_rows, zzd, ones)
    dis, yw1 = _tc_prep(deg.T, x, W1)

    acc1 = _sc_propagate(yw1, src_rows, dst_rows, zz64, 64)
    yw2 = _tc_layer(acc1, yw1, dis, b1.reshape(1, -1), W2)

    acc2 = _sc_propagate(yw2, src_rows, dst_rows, zz32, 32)
    yw3 = _tc_layer(acc2, yw2, dis, b2.reshape(1, -1), W3)

    acc3 = _sc_propagate(yw3, src_rows, dst_rows, zz16, 16)
    h3 = _tc_h3(acc3, yw3, dis, b3.reshape(1, -1))

    h3f = h3.reshape(1, N * 16)
    z = _tc_encoder(h3f, enc1_W, enc1_b.reshape(1, -1), enc2_W,
                    enc2_b.reshape(1, -1), dec1_W, dec1_b.reshape(1, -1))
    return _tc_decoder(z, dec2_W, dec2_b.reshape(1, -1))


# speedup vs baseline: 40.1312x; 1.0162x over previous
import functools

import jax
import jax.numpy as jnp
from jax import lax
from jax.experimental import pallas as pl
from jax.experimental.pallas import tpu as pltpu
from jax.experimental.pallas import tpu_sc as plsc

N = 10000
NPAD = 10240
E = 320000
F_IN = 128
NC = 2
NS = 16
NW = NC * NS
CHUNK = 128
EW = 10240
E_PAD = EW * NW
ROWS_W = EW // CHUNK
NODES_T = NPAD // NS

@functools.cache
def _mesh():
    return plsc.VectorSubcoreMesh(
        core_axis_name="c", subcore_axis_name="s",
        num_cores=NC, num_subcores=NS)


def _sc_degree(dst_rows, zz, ones):
    K = 4

    def body(dst_hbm, zz_hbm, ones_hbm, out_hbm, idx_v, ones_v, acc_sh, sem):
        cid = lax.axis_index("c")
        sid = lax.axis_index("s")
        w = sid * NC + cid
        pltpu.sync_copy(zz_hbm, acc_sh.at[pl.ds(sid * NODES_T, NODES_T)])
        pltpu.sync_copy(ones_hbm, ones_v)
        plsc.subcore_barrier()
        pltpu.sync_copy(dst_hbm.at[pl.ds(w * ROWS_W, ROWS_W)], idx_v)

        def step(s, carry):
            for j in range(K):
                pltpu.async_copy(ones_v, acc_sh.at[idx_v.at[s * K + j]], sem,
                                 add=True)
            for j in range(K):
                pltpu.make_async_copy(ones_v, acc_sh.at[idx_v.at[0]],
                                      sem).wait()
            return carry

        lax.fori_loop(0, ROWS_W // K, step, 0)
        plsc.subcore_barrier()
        pltpu.sync_copy(acc_sh.at[pl.ds(sid * NODES_T, NODES_T)],
                        out_hbm.at[cid, pl.ds(sid * NODES_T, NODES_T)])

    k = pl.kernel(
        body,
        out_type=jax.ShapeDtypeStruct((NC, NPAD), jnp.float32),
        mesh=_mesh(),
        compiler_params=pltpu.CompilerParams(use_tc_tiling_on_sc=False),
        scratch_types=[
            pltpu.VMEM((ROWS_W, CHUNK), jnp.int32),
            pltpu.VMEM((CHUNK,), jnp.float32),
            pltpu.VMEM_SHARED((NPAD,), jnp.float32),
            pltpu.SemaphoreType.DMA,
        ],
    )
    return k(dst_rows, zz, ones)


def _sc_propagate(yw, src_rows, dst_rows, zz, F):
    K = 4 if F >= 64 else 8
    NSUP = ROWS_W // K

    def body(yw_hbm, src_hbm, dst_hbm, zz_hbm, out_hbm,
             sidx_v, didx_v, buf_a, buf_b, acc_sh, sem_ga, sem_gb, sem_s):
        cid = lax.axis_index("c")
        sid = lax.axis_index("s")
        w = sid * NC + cid
        pltpu.sync_copy(zz_hbm, acc_sh.at[pl.ds(sid * NODES_T, NODES_T)])
        plsc.subcore_barrier()
        pltpu.sync_copy(src_hbm.at[pl.ds(w * ROWS_W, ROWS_W)], sidx_v)
        pltpu.sync_copy(dst_hbm.at[pl.ds(w * ROWS_W, ROWS_W)], didx_v)

        def fire(s, buf, sem):
            for j in range(K):
                pltpu.async_copy(yw_hbm.at[sidx_v.at[s * K + j]],
                                 buf.at[pl.ds(j * CHUNK, CHUNK)], sem)

        def drain(buf, sem):
            for j in range(K):
                pltpu.make_async_copy(yw_hbm.at[sidx_v.at[0]],
                                      buf.at[pl.ds(j * CHUNK, CHUNK)],
                                      sem).wait()

        def scat(s, buf):
            for j in range(K):
                pltpu.async_copy(buf.at[pl.ds(j * CHUNK, CHUNK)],
                                 acc_sh.at[didx_v.at[s * K + j]], sem_s,
                                 add=True)
            for j in range(K):
                pltpu.make_async_copy(buf.at[pl.ds(j * CHUNK, CHUNK)],
                                      acc_sh.at[didx_v.at[0]], sem_s).wait()

        fire(0, buf_a, sem_ga)

        def pair(p, carry):
            s0 = 2 * p
            s1 = s0 + 1
            drain(buf_a, sem_ga)
            fire(s1, buf_b, sem_gb)
            scat(s0, buf_a)
            drain(buf_b, sem_gb)
            fire(jnp.minimum(s1 + 1, NSUP - 1), buf_a, sem_ga)
            scat(s1, buf_b)
            return carry

        lax.fori_loop(0, NSUP // 2, pair, 0)
        drain(buf_a, sem_ga)
        plsc.subcore_barrier()
        pltpu.sync_copy(acc_sh.at[pl.ds(sid * NODES_T, NODES_T)],
                        out_hbm.at[cid, pl.ds(sid * NODES_T, NODES_T)])

    k = pl.kernel(
        body,
        out_type=jax.ShapeDtypeStruct((NC, NPAD, F), jnp.float32),
        mesh=_mesh(),
        compiler_params=pltpu.CompilerParams(use_tc_tiling_on_sc=False),
        scratch_types=[
            pltpu.VMEM((ROWS_W, CHUNK), jnp.int32),
            pltpu.VMEM((ROWS_W, CHUNK), jnp.int32),
            pltpu.VMEM((K * CHUNK, F), jnp.float32),
            pltpu.VMEM((K * CHUNK, F), jnp.float32),
            pltpu.VMEM_SHARED((NPAD, F), jnp.float32),
            pltpu.SemaphoreType.DMA,
            pltpu.SemaphoreType.DMA,
            pltpu.SemaphoreType.DMA,
        ],
    )
    return k(yw, src_rows, dst_rows, zz)


_BN = 2000


def _tc_prep(degT, x, W1):
    fo = W1.shape[1]

    def body(deg_ref, x_ref, w_ref, dis_ref, yw_ref):
        d = lax.rsqrt(deg_ref[:, 0:1] + deg_ref[:, 1:2] + 1.0)
        dis_ref[...] = d
        yw_ref[...] = d * jnp.dot(x_ref[...], w_ref[...],
                                  preferred_element_type=jnp.float32)

    return pl.pallas_call(
        body,
        grid=(N // _BN,),
        in_specs=[
            pl.BlockSpec((_BN, 2), lambda i: (i, 0)),
            pl.BlockSpec((_BN, F_IN), lambda i: (i, 0)),
            pl.BlockSpec((F_IN, fo), lambda i: (0, 0)),
        ],
        out_specs=[
            pl.BlockSpec((_BN, 1), lambda i: (i, 0)),
            pl.BlockSpec((_BN, fo), lambda i: (i, 0)),
        ],
        out_shape=[
            jax.ShapeDtypeStruct((N, 1), jnp.float32),
            jax.ShapeDtypeStruct((N, fo), jnp.float32),
        ],
    )(degT, x, W1)


def _tc_layer(acc, yw, dis, b2d, Wn):
    fi, fo = Wn.shape

    def body(acc_ref, yw_ref, dis_ref, b_ref, w_ref, out_ref):
        s = acc_ref[0] + acc_ref[1] + yw_ref[...]
        h = jnp.maximum(dis_ref[...] * s + b_ref[...], 0.0)
        out_ref[...] = dis_ref[...] * jnp.dot(
            h, w_ref[...], preferred_element_type=jnp.float32)

    return pl.pallas_call(
        body,
        grid=(N // _BN,),
        in_specs=[
            pl.BlockSpec((NC, _BN, fi), lambda i: (0, i, 0)),
            pl.BlockSpec((_BN, fi), lambda i: (i, 0)),
            pl.BlockSpec((_BN, 1), lambda i: (i, 0)),
            pl.BlockSpec((1, fi), lambda i: (0, 0)),
            pl.BlockSpec((fi, fo), lambda i: (0, 0)),
        ],
        out_specs=pl.BlockSpec((_BN, fo), lambda i: (i, 0)),
        out_shape=jax.ShapeDtypeStruct((N, fo), jnp.float32),
    )(acc, yw, dis, b2d, Wn)


def _tc_h3(acc, yw, dis, b2d):
    fi = yw.shape[1]
    bn = N

    def body(acc_ref, yw_ref, dis_ref, b_ref, out_ref):
        s = acc_ref[0] + acc_ref[1] + yw_ref[...]
        out_ref[...] = dis_ref[...] * s + b_ref[...]

    return pl.pallas_call(
        body,
        grid=(1,),
        in_specs=[
            pl.BlockSpec((NC, bn, fi), lambda i: (0, i, 0)),
            pl.BlockSpec((bn, fi), lambda i: (i, 0)),
            pl.BlockSpec((bn, 1), lambda i: (i, 0)),
            pl.BlockSpec((1, fi), lambda i: (0, 0)),
        ],
        out_specs=pl.BlockSpec((bn, fi), lambda i: (i, 0)),
        out_shape=jax.ShapeDtypeStruct((N, fi), jnp.float32),
    )(acc, yw, dis, b2d)


def _tc_encoder(h3f, enc1_W, enc1_b, enc2_W, enc2_b, dec1_W, dec1_b):
    K = h3f.shape[1]
    BK = 16000
    steps = K // BK

    def body(h_ref, w_ref, b1_ref, w2_ref, b2_ref, w3_ref, b3_ref,
             out_ref, acc_ref):
        i = pl.program_id(0)
        part = jnp.dot(h_ref[...], w_ref[...],
                       preferred_element_type=jnp.float32)

        @pl.when(i == 0)
        def _():
            acc_ref[...] = part

        @pl.when(i > 0)
        def _():
            acc_ref[...] = acc_ref[...] + part

        @pl.when(i == steps - 1)
        def _():
            z = jnp.maximum(acc_ref[...] + b1_ref[...], 0.0)
            z = jnp.dot(z, w2_ref[...],
                        preferred_element_type=jnp.float32) + b2_ref[...]
            z = jnp.maximum(
                jnp.dot(z, w3_ref[...],
                        preferred_element_type=jnp.float32) + b3_ref[...],
                0.0)
            out_ref[...] = z

    return pl.pallas_call(
        body,
        grid=(steps,),
        in_specs=[
            pl.BlockSpec((1, BK), lambda i: (0, i)),
            pl.BlockSpec((BK, 128), lambda i: (i, 0)),
            pl.BlockSpec((1, 128), lambda i: (0, 0)),
            pl.BlockSpec((128, 64), lambda i: (0, 0)),
            pl.BlockSpec((1, 64), lambda i: (0, 0)),
            pl.BlockSpec((64, 128), lambda i: (0, 0)),
            pl.BlockSpec((1, 128), lambda i: (0, 0)),
        ],
        out_specs=pl.BlockSpec((1, 128), lambda i: (0, 0)),
        out_shape=jax.ShapeDtypeStruct((1, 128), jnp.float32),
        scratch_shapes=[pltpu.VMEM((1, 128), jnp.float32)],
    )(h3f, enc1_W, enc1_b, enc2_W, enc2_b, dec1_W, dec1_b)


def _tc_decoder(z, dec2_W, dec2_b):
    def body(z_ref, w_ref, b_ref, out_ref):
        t = jnp.dot(z_ref[...], w_ref[...],
                    preferred_element_type=jnp.float32) + b_ref[...]
        out_ref[...] = 1.0 / (1.0 + jnp.exp(-t))

    return pl.pallas_call(
        body,
        out_shape=jax.ShapeDtypeStruct((1, N), jnp.float32),
    )(z, dec2_W, dec2_b)


def kernel(x, edge_index, batch_size, batch_index, W1, b1, W2, b2, W3, b3,
           enc1_W, enc1_b, enc2_W, enc2_b, dec1_W, dec1_b, dec2_W, dec2_b):
    del batch_size, batch_index
    pad = jnp.arange(E_PAD - E, dtype=jnp.int32) % (NPAD - N)
    src_rows = jnp.concatenate([edge_index[0], pad]).reshape(E_PAD // CHUNK,
                                                             CHUNK)
    dst_rows = jnp.concatenate([edge_index[1], pad + N]).reshape(
        E_PAD // CHUNK, CHUNK)

    zzd = jnp.zeros((NODES_T,), jnp.float32)
    zz64 = jnp.zeros((NODES_T, 64), jnp.float32)
    zz32 = jnp.zeros((NODES_T, 32), jnp.float32)
    zz16 = jnp.zeros((NODES_T, 16), jnp.float32)
    ones = jnp.ones((CHUNK,), jnp.float32)

    deg = _sc_degree(dst_rows, zzd, ones)
    dis, yw1 = _tc_prep(deg.T, x, W1)

    acc1 = _sc_propagate(yw1, src_rows, dst_rows, zz64, 64)
    yw2 = _tc_layer(acc1, yw1, dis, b1.reshape(1, -1), W2)

    acc2 = _sc_propagate(yw2, src_rows, dst_rows, zz32, 32)
    yw3 = _tc_layer(acc2, yw2, dis, b2.reshape(1, -1), W3)

    acc3 = _sc_propagate(yw3, src_rows, dst_rows, zz16, 16)
    h3 = _tc_h3(acc3, yw3, dis, b3.reshape(1, -1))

    h3f = h3.reshape(1, N * 16)
    z = _tc_encoder(h3f, enc1_W, enc1_b.reshape(1, -1), enc2_W,
                    enc2_b.reshape(1, -1), dec1_W, dec1_b.reshape(1, -1))
    return _tc_decoder(z, dec2_W, dec2_b.reshape(1, -1))


# speedup vs baseline: 40.4390x; 1.0077x over previous
import functools

import jax
import jax.numpy as jnp
from jax import lax
from jax.experimental import pallas as pl
from jax.experimental.pallas import tpu as pltpu
from jax.experimental.pallas import tpu_sc as plsc

N = 10000
NPAD = 10240
E = 320000
F_IN = 128
NC = 2
NS = 16
NW = NC * NS
CHUNK = 128
EW = 10240
E_PAD = EW * NW
ROWS_W = EW // CHUNK
NODES_T = NPAD // NS

@functools.cache
def _mesh():
    return plsc.VectorSubcoreMesh(
        core_axis_name="c", subcore_axis_name="s",
        num_cores=NC, num_subcores=NS)


def _sc_degree(dst_rows, zz, ones):
    K = 4

    def body(dst_hbm, zz_hbm, ones_hbm, out_hbm, idx_v, ones_v, acc_sh, sem):
        cid = lax.axis_index("c")
        sid = lax.axis_index("s")
        w = sid * NC + cid
        pltpu.sync_copy(zz_hbm, acc_sh.at[pl.ds(sid * NODES_T, NODES_T)])
        pltpu.sync_copy(ones_hbm, ones_v)
        plsc.subcore_barrier()
        pltpu.sync_copy(dst_hbm.at[pl.ds(w * ROWS_W, ROWS_W)], idx_v)

        def step(s, carry):
            for j in range(K):
                pltpu.async_copy(ones_v, acc_sh.at[idx_v.at[s * K + j]], sem,
                                 add=True)
            for j in range(K):
                pltpu.make_async_copy(ones_v, acc_sh.at[idx_v.at[0]],
                                      sem).wait()
            return carry

        lax.fori_loop(0, ROWS_W // K, step, 0)
        plsc.subcore_barrier()
        pltpu.sync_copy(acc_sh.at[pl.ds(sid * NODES_T, NODES_T)],
                        out_hbm.at[cid, pl.ds(sid * NODES_T, NODES_T)])

    k = pl.kernel(
        body,
        out_type=jax.ShapeDtypeStruct((NC, NPAD), jnp.float32),
        mesh=_mesh(),
        compiler_params=pltpu.CompilerParams(use_tc_tiling_on_sc=False),
        scratch_types=[
            pltpu.VMEM((ROWS_W, CHUNK), jnp.int32),
            pltpu.VMEM((CHUNK,), jnp.float32),
            pltpu.VMEM_SHARED((NPAD,), jnp.float32),
            pltpu.SemaphoreType.DMA,
        ],
    )
    return k(dst_rows, zz, ones)


def _sc_propagate(yw, src_rows, dst_rows, zz, F):
    K = {64: 4, 32: 10, 16: 20}[F]
    NSUP = ROWS_W // K

    def body(yw_hbm, src_hbm, dst_hbm, zz_hbm, out_hbm,
             sidx_v, didx_v, buf_a, buf_b, acc_sh, sem_ga, sem_gb, sem_s):
        cid = lax.axis_index("c")
        sid = lax.axis_index("s")
        w = sid * NC + cid
        pltpu.sync_copy(zz_hbm, acc_sh.at[pl.ds(sid * NODES_T, NODES_T)])
        plsc.subcore_barrier()
        pltpu.sync_copy(src_hbm.at[pl.ds(w * ROWS_W, ROWS_W)], sidx_v)
        pltpu.sync_copy(dst_hbm.at[pl.ds(w * ROWS_W, ROWS_W)], didx_v)

        def fire(s, buf, sem):
            for j in range(K):
                pltpu.async_copy(yw_hbm.at[sidx_v.at[s * K + j]],
                                 buf.at[pl.ds(j * CHUNK, CHUNK)], sem)

        def drain(buf, sem):
            for j in range(K):
                pltpu.make_async_copy(yw_hbm.at[sidx_v.at[0]],
                                      buf.at[pl.ds(j * CHUNK, CHUNK)],
                                      sem).wait()

        def scat(s, buf):
            for j in range(K):
                pltpu.async_copy(buf.at[pl.ds(j * CHUNK, CHUNK)],
                                 acc_sh.at[didx_v.at[s * K + j]], sem_s,
                                 add=True)
            for j in range(K):
                pltpu.make_async_copy(buf.at[pl.ds(j * CHUNK, CHUNK)],
                                      acc_sh.at[didx_v.at[0]], sem_s).wait()

        fire(0, buf_a, sem_ga)

        def pair(p, carry):
            s0 = 2 * p
            s1 = s0 + 1
            drain(buf_a, sem_ga)
            fire(s1, buf_b, sem_gb)
            scat(s0, buf_a)
            drain(buf_b, sem_gb)
            fire(jnp.minimum(s1 + 1, NSUP - 1), buf_a, sem_ga)
            scat(s1, buf_b)
            return carry

        lax.fori_loop(0, NSUP // 2, pair, 0)
        drain(buf_a, sem_ga)
        plsc.subcore_barrier()
        pltpu.sync_copy(acc_sh.at[pl.ds(sid * NODES_T, NODES_T)],
                        out_hbm.at[cid, pl.ds(sid * NODES_T, NODES_T)])

    k = pl.kernel(
        body,
        out_type=jax.ShapeDtypeStruct((NC, NPAD, F), jnp.float32),
        mesh=_mesh(),
        compiler_params=pltpu.CompilerParams(use_tc_tiling_on_sc=False),
        scratch_types=[
            pltpu.VMEM((ROWS_W, CHUNK), jnp.int32),
            pltpu.VMEM((ROWS_W, CHUNK), jnp.int32),
            pltpu.VMEM((K * CHUNK, F), jnp.float32),
            pltpu.VMEM((K * CHUNK, F), jnp.float32),
            pltpu.VMEM_SHARED((NPAD, F), jnp.float32),
            pltpu.SemaphoreType.DMA,
            pltpu.SemaphoreType.DMA,
            pltpu.SemaphoreType.DMA,
        ],
    )
    return k(yw, src_rows, dst_rows, zz)


_BN = 2000


def _tc_prep(degT, x, W1):
    fo = W1.shape[1]

    def body(deg_ref, x_ref, w_ref, dis_ref, yw_ref):
        d = lax.rsqrt(deg_ref[:, 0:1] + deg_ref[:, 1:2] + 1.0)
        dis_ref[...] = d
        yw_ref[...] = d * jnp.dot(x_ref[...], w_ref[...],
                                  preferred_element_type=jnp.float32)

    return pl.pallas_call(
        body,
        grid=(N // _BN,),
        in_specs=[
            pl.BlockSpec((_BN, 2), lambda i: (i, 0)),
            pl.BlockSpec((_BN, F_IN), lambda i: (i, 0)),
            pl.BlockSpec((F_IN, fo), lambda i: (0, 0)),
        ],
        out_specs=[
            pl.BlockSpec((_BN, 1), lambda i: (i, 0)),
            pl.BlockSpec((_BN, fo), lambda i: (i, 0)),
        ],
        out_shape=[
            jax.ShapeDtypeStruct((N, 1), jnp.float32),
            jax.ShapeDtypeStruct((N, fo), jnp.float32),
        ],
    )(degT, x, W1)


def _tc_layer(acc, yw, dis, b2d, Wn):
    fi, fo = Wn.shape

    def body(acc_ref, yw_ref, dis_ref, b_ref, w_ref, out_ref):
        s = acc_ref[0] + acc_ref[1] + yw_ref[...]
        h = jnp.maximum(dis_ref[...] * s + b_ref[...], 0.0)
        out_ref[...] = dis_ref[...] * jnp.dot(
            h, w_ref[...], preferred_element_type=jnp.float32)

    return pl.pallas_call(
        body,
        grid=(N // _BN,),
        in_specs=[
            pl.BlockSpec((NC, _BN, fi), lambda i: (0, i, 0)),
            pl.BlockSpec((_BN, fi), lambda i: (i, 0)),
            pl.BlockSpec((_BN, 1), lambda i: (i, 0)),
            pl.BlockSpec((1, fi), lambda i: (0, 0)),
            pl.BlockSpec((fi, fo), lambda i: (0, 0)),
        ],
        out_specs=pl.BlockSpec((_BN, fo), lambda i: (i, 0)),
        out_shape=jax.ShapeDtypeStruct((N, fo), jnp.float32),
    )(acc, yw, dis, b2d, Wn)


def _tc_h3(acc, yw, dis, b2d):
    fi = yw.shape[1]
    bn = N

    def body(acc_ref, yw_ref, dis_ref, b_ref, out_ref):
        s = acc_ref[0] + acc_ref[1] + yw_ref[...]
        out_ref[...] = dis_ref[...] * s + b_ref[...]

    return pl.pallas_call(
        body,
        grid=(1,),
        in_specs=[
            pl.BlockSpec((NC, bn, fi), lambda i: (0, i, 0)),
            pl.BlockSpec((bn, fi), lambda i: (i, 0)),
            pl.BlockSpec((bn, 1), lambda i: (i, 0)),
            pl.BlockSpec((1, fi), lambda i: (0, 0)),
        ],
        out_specs=pl.BlockSpec((bn, fi), lambda i: (i, 0)),
        out_shape=jax.ShapeDtypeStruct((N, fi), jnp.float32),
    )(acc, yw, dis, b2d)


def _tc_encoder(h3f, enc1_W, enc1_b, enc2_W, enc2_b, dec1_W, dec1_b):
    K = h3f.shape[1]
    BK = 16000
    steps = K // BK

    def body(h_ref, w_ref, b1_ref, w2_ref, b2_ref, w3_ref, b3_ref,
             out_ref, acc_ref):
        i = pl.program_id(0)
        part = jnp.dot(h_ref[...], w_ref[...],
                       preferred_element_type=jnp.float32)

        @pl.when(i == 0)
        def _():
            acc_ref[...] = part

        @pl.when(i > 0)
        def _():
            acc_ref[...] = acc_ref[...] + part

        @pl.when(i == steps - 1)
        def _():
            z = jnp.maximum(acc_ref[...] + b1_ref[...], 0.0)
            z = jnp.dot(z, w2_ref[...],
                        preferred_element_type=jnp.float32) + b2_ref[...]
            z = jnp.maximum(
                jnp.dot(z, w3_ref[...],
                        preferred_element_type=jnp.float32) + b3_ref[...],
                0.0)
            out_ref[...] = z

    return pl.pallas_call(
        body,
        grid=(steps,),
        in_specs=[
            pl.BlockSpec((1, BK), lambda i: (0, i)),
            pl.BlockSpec((BK, 128), lambda i: (i, 0)),
            pl.BlockSpec((1, 128), lambda i: (0, 0)),
            pl.BlockSpec((128, 64), lambda i: (0, 0)),
            pl.BlockSpec((1, 64), lambda i: (0, 0)),
            pl.BlockSpec((64, 128), lambda i: (0, 0)),
            pl.BlockSpec((1, 128), lambda i: (0, 0)),
        ],
        out_specs=pl.BlockSpec((1, 128), lambda i: (0, 0)),
        out_shape=jax.ShapeDtypeStruct((1, 128), jnp.float32),
        scratch_shapes=[pltpu.VMEM((1, 128), jnp.float32)],
    )(h3f, enc1_W, enc1_b, enc2_W, enc2_b, dec1_W, dec1_b)


def _tc_decoder(z, dec2_W, dec2_b):
    def body(z_ref, w_ref, b_ref, out_ref):
        t = jnp.dot(z_ref[...], w_ref[...],
                    preferred_element_type=jnp.float32) + b_ref[...]
        out_ref[...] = 1.0 / (1.0 + jnp.exp(-t))

    return pl.pallas_call(
        body,
        out_shape=jax.ShapeDtypeStruct((1, N), jnp.float32),
    )(z, dec2_W, dec2_b)


def kernel(x, edge_index, batch_size, batch_index, W1, b1, W2, b2, W3, b3,
           enc1_W, enc1_b, enc2_W, enc2_b, dec1_W, dec1_b, dec2_W, dec2_b):
    del batch_size, batch_index
    pad = jnp.arange(E_PAD - E, dtype=jnp.int32) % (NPAD - N)
    src_rows = jnp.concatenate([edge_index[0], pad]).reshape(E_PAD // CHUNK,
                                                             CHUNK)
    dst_rows = jnp.concatenate([edge_index[1], pad + N]).reshape(
        E_PAD // CHUNK, CHUNK)

    zzd = jnp.zeros((NODES_T,), jnp.float32)
    zz64 = jnp.zeros((NODES_T, 64), jnp.float32)
    zz32 = jnp.zeros((NODES_T, 32), jnp.float32)
    zz16 = jnp.zeros((NODES_T, 16), jnp.float32)
    ones = jnp.ones((CHUNK,), jnp.float32)

    deg = _sc_degree(dst_rows, zzd, ones)
    dis, yw1 = _tc_prep(deg.T, x, W1)

    acc1 = _sc_propagate(yw1, src_rows, dst_rows, zz64, 64)
    yw2 = _tc_layer(acc1, yw1, dis, b1.reshape(1, -1), W2)

    acc2 = _sc_propagate(yw2, src_rows, dst_rows, zz32, 32)
    yw3 = _tc_layer(acc2, yw2, dis, b2.reshape(1, -1), W3)

    acc3 = _sc_propagate(yw3, src_rows, dst_rows, zz16, 16)
    h3 = _tc_h3(acc3, yw3, dis, b3.reshape(1, -1))

    h3f = h3.reshape(1, N * 16)
    z = _tc_encoder(h3f, enc1_W, enc1_b.reshape(1, -1), enc2_W,
                    enc2_b.reshape(1, -1), dec1_W, dec1_b.reshape(1, -1))
    return _tc_decoder(z, dec2_W, dec2_b.reshape(1, -1))


# speedup vs baseline: 40.6412x; 1.0050x over previous
import functools

import jax
import jax.numpy as jnp
from jax import lax
from jax.experimental import pallas as pl
from jax.experimental.pallas import tpu as pltpu
from jax.experimental.pallas import tpu_sc as plsc

N = 10000
NPAD = 10240
E = 320000
F_IN = 128
NC = 2
NS = 16
NW = NC * NS
CHUNK = 128
EW = 10240
E_PAD = EW * NW
ROWS_W = EW // CHUNK
NODES_T = NPAD // NS

@functools.cache
def _mesh():
    return plsc.VectorSubcoreMesh(
        core_axis_name="c", subcore_axis_name="s",
        num_cores=NC, num_subcores=NS)


def _sc_degree(dst_rows, zz, ones):
    K = 4

    def body(dst_hbm, zz_hbm, ones_hbm, out_hbm, idx_v, ones_v, acc_sh, sem):
        cid = lax.axis_index("c")
        sid = lax.axis_index("s")
        w = sid * NC + cid
        pltpu.sync_copy(zz_hbm, acc_sh.at[pl.ds(sid * NODES_T, NODES_T)])
        pltpu.sync_copy(ones_hbm, ones_v)
        plsc.subcore_barrier()
        pltpu.sync_copy(dst_hbm.at[pl.ds(w * ROWS_W, ROWS_W)], idx_v)

        def step(s, carry):
            for j in range(K):
                pltpu.async_copy(ones_v, acc_sh.at[idx_v.at[s * K + j]], sem,
                                 add=True)
            for j in range(K):
                pltpu.make_async_copy(ones_v, acc_sh.at[idx_v.at[0]],
                                      sem).wait()
            return carry

        lax.fori_loop(0, ROWS_W // K, step, 0)
        plsc.subcore_barrier()
        pltpu.sync_copy(acc_sh.at[pl.ds(sid * NODES_T, NODES_T)],
                        out_hbm.at[cid, pl.ds(sid * NODES_T, NODES_T)])

    k = pl.kernel(
        body,
        out_type=jax.ShapeDtypeStruct((NC, NPAD), jnp.float32),
        mesh=_mesh(),
        compiler_params=pltpu.CompilerParams(use_tc_tiling_on_sc=False),
        scratch_types=[
            pltpu.VMEM((ROWS_W, CHUNK), jnp.int32),
            pltpu.VMEM((CHUNK,), jnp.float32),
            pltpu.VMEM_SHARED((NPAD,), jnp.float32),
            pltpu.SemaphoreType.DMA,
        ],
    )
    return k(dst_rows, zz, ones)


def _sc_propagate(yw, src_rows, dst_rows, zz, F):
    K = {64: 4, 32: 10, 16: 20}[F]
    NSUP = ROWS_W // K

    def body(yw_hbm, src_hbm, dst_hbm, zz_hbm, out_hbm,
             sidx_v, didx_v, buf_a, buf_b, acc_sh, sem_ga, sem_gb, sem_s):
        cid = lax.axis_index("c")
        sid = lax.axis_index("s")
        w = sid * NC + cid
        pltpu.sync_copy(zz_hbm, acc_sh.at[pl.ds(sid * NODES_T, NODES_T)])
        plsc.subcore_barrier()
        pltpu.sync_copy(src_hbm.at[pl.ds(w * ROWS_W, ROWS_W)], sidx_v)
        pltpu.sync_copy(dst_hbm.at[pl.ds(w * ROWS_W, ROWS_W)], didx_v)

        def fire(s, buf, sem):
            for j in range(K):
                pltpu.async_copy(yw_hbm.at[sidx_v.at[s * K + j]],
                                 buf.at[pl.ds(j * CHUNK, CHUNK)], sem)

        def drain(buf, sem):
            for j in range(K):
                pltpu.make_async_copy(yw_hbm.at[sidx_v.at[0]],
                                      buf.at[pl.ds(j * CHUNK, CHUNK)],
                                      sem).wait()

        def scat(s, buf):
            for j in range(K):
                pltpu.async_copy(buf.at[pl.ds(j * CHUNK, CHUNK)],
                                 acc_sh.at[didx_v.at[s * K + j]], sem_s,
                                 add=True)
            for j in range(K):
                pltpu.make_async_copy(buf.at[pl.ds(j * CHUNK, CHUNK)],
                                      acc_sh.at[didx_v.at[0]], sem_s).wait()

        fire(0, buf_a, sem_ga)

        def pair(p, carry):
            s0 = 2 * p
            s1 = s0 + 1
            drain(buf_a, sem_ga)
            fire(s1, buf_b, sem_gb)
            scat(s0, buf_a)
            drain(buf_b, sem_gb)
            fire(jnp.minimum(s1 + 1, NSUP - 1), buf_a, sem_ga)
            scat(s1, buf_b)
            return carry

        lax.fori_loop(0, NSUP // 2, pair, 0)
        drain(buf_a, sem_ga)
        plsc.subcore_barrier()
        pltpu.sync_copy(acc_sh.at[pl.ds(sid * NODES_T, NODES_T)],
                        out_hbm.at[cid, pl.ds(sid * NODES_T, NODES_T)])

    k = pl.kernel(
        body,
        out_type=jax.ShapeDtypeStruct((NC, NPAD, F), jnp.float32),
        mesh=_mesh(),
        compiler_params=pltpu.CompilerParams(use_tc_tiling_on_sc=False),
        scratch_types=[
            pltpu.VMEM((ROWS_W, CHUNK), jnp.int32),
            pltpu.VMEM((ROWS_W, CHUNK), jnp.int32),
            pltpu.VMEM((K * CHUNK, F), jnp.float32),
            pltpu.VMEM((K * CHUNK, F), jnp.float32),
            pltpu.VMEM_SHARED((NPAD, F), jnp.float32),
            pltpu.SemaphoreType.DMA,
            pltpu.SemaphoreType.DMA,
            pltpu.SemaphoreType.DMA,
        ],
    )
    return k(yw, src_rows, dst_rows, zz)


_BN = 2000


def _tc_prep(degT, x, W1):
    fo = W1.shape[1]

    def body(deg_ref, x_ref, w_ref, dis_ref, yw_ref):
        d = lax.rsqrt(deg_ref[:, 0:1] + deg_ref[:, 1:2] + 1.0)
        dis_ref[...] = d
        yw_ref[...] = d * jnp.dot(x_ref[...], w_ref[...],
                                  preferred_element_type=jnp.float32)

    return pl.pallas_call(
        body,
        grid=(N // _BN,),
        in_specs=[
            pl.BlockSpec((_BN, 2), lambda i: (i, 0)),
            pl.BlockSpec((_BN, F_IN), lambda i: (i, 0)),
            pl.BlockSpec((F_IN, fo), lambda i: (0, 0)),
        ],
        out_specs=[
            pl.BlockSpec((_BN, 1), lambda i: (i, 0)),
            pl.BlockSpec((_BN, fo), lambda i: (i, 0)),
        ],
        out_shape=[
            jax.ShapeDtypeStruct((N, 1), jnp.float32),
            jax.ShapeDtypeStruct((N, fo), jnp.float32),
        ],
    )(degT, x, W1)


def _tc_layer(acc, yw, dis, b2d, Wn):
    fi, fo = Wn.shape

    def body(acc_ref, yw_ref, dis_ref, b_ref, w_ref, out_ref):
        s = acc_ref[0] + acc_ref[1] + yw_ref[...]
        h = jnp.maximum(dis_ref[...] * s + b_ref[...], 0.0)
        out_ref[...] = dis_ref[...] * jnp.dot(
            h, w_ref[...], preferred_element_type=jnp.float32)

    return pl.pallas_call(
        body,
        grid=(N // _BN,),
        in_specs=[
            pl.BlockSpec((NC, _BN, fi), lambda i: (0, i, 0)),
            pl.BlockSpec((_BN, fi), lambda i: (i, 0)),
            pl.BlockSpec((_BN, 1), lambda i: (i, 0)),
            pl.BlockSpec((1, fi), lambda i: (0, 0)),
            pl.BlockSpec((fi, fo), lambda i: (0, 0)),
        ],
        out_specs=pl.BlockSpec((_BN, fo), lambda i: (i, 0)),
        out_shape=jax.ShapeDtypeStruct((N, fo), jnp.float32),
    )(acc, yw, dis, b2d, Wn)


def _tc_h3(acc, yw, dis, b2d):
    fi = yw.shape[1]
    bn = N

    def body(acc_ref, yw_ref, dis_ref, b_ref, out_ref):
        s = acc_ref[0] + acc_ref[1] + yw_ref[...]
        out_ref[...] = dis_ref[...] * s + b_ref[...]

    return pl.pallas_call(
        body,
        grid=(1,),
        in_specs=[
            pl.BlockSpec((NC, bn, fi), lambda i: (0, i, 0)),
            pl.BlockSpec((bn, fi), lambda i: (i, 0)),
            pl.BlockSpec((bn, 1), lambda i: (i, 0)),
            pl.BlockSpec((1, fi), lambda i: (0, 0)),
        ],
        out_specs=pl.BlockSpec((bn, fi), lambda i: (i, 0)),
        out_shape=jax.ShapeDtypeStruct((N, fi), jnp.float32),
    )(acc, yw, dis, b2d)


def _tc_encdec(h3f, enc1_W, enc1_b, enc2_W, enc2_b, dec1_W, dec1_b,
               dec2_W, dec2_b):
    K = h3f.shape[1]
    BK = 16000
    steps = K // BK

    def body(h_ref, w_ref, b1_ref, w2_ref, b2_ref, w3_ref, b3_ref,
             w4_ref, b4_ref, out_ref, acc_ref):
        i = pl.program_id(0)
        part = jnp.dot(h_ref[...], w_ref[...],
                       preferred_element_type=jnp.float32)

        @pl.when(i == 0)
        def _():
            acc_ref[...] = part

        @pl.when(i > 0)
        def _():
            acc_ref[...] = acc_ref[...] + part

        @pl.when(i == steps - 1)
        def _():
            z = jnp.maximum(acc_ref[...] + b1_ref[...], 0.0)
            z = jnp.dot(z, w2_ref[...],
                        preferred_element_type=jnp.float32) + b2_ref[...]
            z = jnp.maximum(
                jnp.dot(z, w3_ref[...],
                        preferred_element_type=jnp.float32) + b3_ref[...],
                0.0)
            t = jnp.dot(z, w4_ref[...],
                        preferred_element_type=jnp.float32) + b4_ref[...]
            out_ref[...] = 1.0 / (1.0 + jnp.exp(-t))

    return pl.pallas_call(
        body,
        grid=(steps,),
        in_specs=[
            pl.BlockSpec((1, BK), lambda i: (0, i)),
            pl.BlockSpec((BK, 128), lambda i: (i, 0)),
            pl.BlockSpec((1, 128), lambda i: (0, 0)),
            pl.BlockSpec((128, 64), lambda i: (0, 0)),
            pl.BlockSpec((1, 64), lambda i: (0, 0)),
            pl.BlockSpec((64, 128), lambda i: (0, 0)),
            pl.BlockSpec((1, 128), lambda i: (0, 0)),
            pl.BlockSpec((128, N), lambda i: (0, 0)),
            pl.BlockSpec((1, N), lambda i: (0, 0)),
        ],
        out_specs=pl.BlockSpec((1, N), lambda i: (0, 0)),
        out_shape=jax.ShapeDtypeStruct((1, N), jnp.float32),
        scratch_shapes=[pltpu.VMEM((1, 128), jnp.float32)],
    )(h3f, enc1_W, enc1_b, enc2_W, enc2_b, dec1_W, dec1_b, dec2_W, dec2_b)


def kernel(x, edge_index, batch_size, batch_index, W1, b1, W2, b2, W3, b3,
           enc1_W, enc1_b, enc2_W, enc2_b, dec1_W, dec1_b, dec2_W, dec2_b):
    del batch_size, batch_index
    pad = jnp.arange(E_PAD - E, dtype=jnp.int32) % (NPAD - N)
    src_rows = jnp.concatenate([edge_index[0], pad]).reshape(E_PAD // CHUNK,
                                                             CHUNK)
    dst_rows = jnp.concatenate([edge_index[1], pad + N]).reshape(
        E_PAD // CHUNK, CHUNK)

    zzd = jnp.zeros((NODES_T,), jnp.float32)
    zz64 = jnp.zeros((NODES_T, 64), jnp.float32)
    zz32 = jnp.zeros((NODES_T, 32), jnp.float32)
    zz16 = jnp.zeros((NODES_T, 16), jnp.float32)
    ones = jnp.ones((CHUNK,), jnp.float32)

    deg = _sc_degree(dst_rows, zzd, ones)
    dis, yw1 = _tc_prep(deg.T, x, W1)

    acc1 = _sc_propagate(yw1, src_rows, dst_rows, zz64, 64)
    yw2 = _tc_layer(acc1, yw1, dis, b1.reshape(1, -1), W2)

    acc2 = _sc_propagate(yw2, src_rows, dst_rows, zz32, 32)
    yw3 = _tc_layer(acc2, yw2, dis, b2.reshape(1, -1), W3)

    acc3 = _sc_propagate(yw3, src_rows, dst_rows, zz16, 16)
    h3 = _tc_h3(acc3, yw3, dis, b3.reshape(1, -1))

    h3f = h3.reshape(1, N * 16)
    return _tc_encdec(h3f, enc1_W, enc1_b.reshape(1, -1), enc2_W,
                      enc2_b.reshape(1, -1), dec1_W, dec1_b.reshape(1, -1),
                      dec2_W, dec2_b.reshape(1, -1))


# speedup vs baseline: 42.1816x; 1.0379x over previous
import functools

import jax
import jax.numpy as jnp
from jax import lax
from jax.experimental import pallas as pl
from jax.experimental.pallas import tpu as pltpu
from jax.experimental.pallas import tpu_sc as plsc

N = 10000
NPAD = 10240
E = 320000
F_IN = 128
NC = 2
NS = 16
NW = NC * NS
CHUNK = 128
EW = 10240
E_PAD = EW * NW
ROWS_W = EW // CHUNK
NODES_T = NPAD // NS

@functools.cache
def _mesh():
    return plsc.VectorSubcoreMesh(
        core_axis_name="c", subcore_axis_name="s",
        num_cores=NC, num_subcores=NS)


def _sc_degree(edges, zz, ones):
    K = 4

    def body(edges_hbm, zz_hbm, ones_hbm, out_hbm, idx_v, ones_v, acc_sh, sem):
        cid = lax.axis_index("c")
        sid = lax.axis_index("s")
        w = sid * NC + cid
        pltpu.sync_copy(zz_hbm, acc_sh.at[pl.ds(sid * NODES_T, NODES_T)])
        pltpu.sync_copy(ones_hbm, ones_v)
        plsc.subcore_barrier()
        pltpu.sync_copy(edges_hbm.at[1, pl.ds(w * ROWS_W, ROWS_W)], idx_v)

        def step(s, carry):
            for j in range(K):
                pltpu.async_copy(ones_v, acc_sh.at[idx_v.at[s * K + j]], sem,
                                 add=True)
            for j in range(K):
                pltpu.make_async_copy(ones_v, acc_sh.at[idx_v.at[0]],
                                      sem).wait()
            return carry

        lax.fori_loop(0, ROWS_W // K, step, 0)
        plsc.subcore_barrier()
        pltpu.sync_copy(acc_sh.at[pl.ds(sid * NODES_T, NODES_T)],
                        out_hbm.at[cid, pl.ds(sid * NODES_T, NODES_T)])

    k = pl.kernel(
        body,
        out_type=jax.ShapeDtypeStruct((NC, NPAD), jnp.float32),
        mesh=_mesh(),
        compiler_params=pltpu.CompilerParams(use_tc_tiling_on_sc=False),
        scratch_types=[
            pltpu.VMEM((ROWS_W, CHUNK), jnp.int32),
            pltpu.VMEM((CHUNK,), jnp.float32),
            pltpu.VMEM_SHARED((NPAD,), jnp.float32),
            pltpu.SemaphoreType.DMA,
        ],
    )
    return k(edges, zz, ones)


def _sc_propagate(yw, edges, zz, F):
    K = {64: 4, 32: 10, 16: 20}[F]
    NSUP = ROWS_W // K

    def body(yw_hbm, edges_hbm, zz_hbm, out_hbm,
             sidx_v, didx_v, buf_a, buf_b, acc_sh, sem_ga, sem_gb, sem_s):
        cid = lax.axis_index("c")
        sid = lax.axis_index("s")
        w = sid * NC + cid
        pltpu.sync_copy(zz_hbm, acc_sh.at[pl.ds(sid * NODES_T, NODES_T)])
        plsc.subcore_barrier()
        pltpu.sync_copy(edges_hbm.at[0, pl.ds(w * ROWS_W, ROWS_W)], sidx_v)
        pltpu.sync_copy(edges_hbm.at[1, pl.ds(w * ROWS_W, ROWS_W)], didx_v)

        def fire(s, buf, sem):
            for j in range(K):
                pltpu.async_copy(yw_hbm.at[sidx_v.at[s * K + j]],
                                 buf.at[pl.ds(j * CHUNK, CHUNK)], sem)

        def drain(buf, sem):
            for j in range(K):
                pltpu.make_async_copy(yw_hbm.at[sidx_v.at[0]],
                                      buf.at[pl.ds(j * CHUNK, CHUNK)],
                                      sem).wait()

        def scat(s, buf):
            for j in range(K):
                pltpu.async_copy(buf.at[pl.ds(j * CHUNK, CHUNK)],
                                 acc_sh.at[didx_v.at[s * K + j]], sem_s,
                                 add=True)
            for j in range(K):
                pltpu.make_async_copy(buf.at[pl.ds(j * CHUNK, CHUNK)],
                                      acc_sh.at[didx_v.at[0]], sem_s).wait()

        fire(0, buf_a, sem_ga)

        def pair(p, carry):
            s0 = 2 * p
            s1 = s0 + 1
            drain(buf_a, sem_ga)
            fire(s1, buf_b, sem_gb)
            scat(s0, buf_a)
            drain(buf_b, sem_gb)
            fire(jnp.minimum(s1 + 1, NSUP - 1), buf_a, sem_ga)
            scat(s1, buf_b)
            return carry

        lax.fori_loop(0, NSUP // 2, pair, 0)
        drain(buf_a, sem_ga)
        plsc.subcore_barrier()
        pltpu.sync_copy(acc_sh.at[pl.ds(sid * NODES_T, NODES_T)],
                        out_hbm.at[cid, pl.ds(sid * NODES_T, NODES_T)])

    k = pl.kernel(
        body,
        out_type=jax.ShapeDtypeStruct((NC, NPAD, F), jnp.float32),
        mesh=_mesh(),
        compiler_params=pltpu.CompilerParams(use_tc_tiling_on_sc=False),
        scratch_types=[
            pltpu.VMEM((ROWS_W, CHUNK), jnp.int32),
            pltpu.VMEM((ROWS_W, CHUNK), jnp.int32),
            pltpu.VMEM((K * CHUNK, F), jnp.float32),
            pltpu.VMEM((K * CHUNK, F), jnp.float32),
            pltpu.VMEM_SHARED((NPAD, F), jnp.float32),
            pltpu.SemaphoreType.DMA,
            pltpu.SemaphoreType.DMA,
            pltpu.SemaphoreType.DMA,
        ],
    )
    return k(yw, edges, zz)


_BN = 2000


def _tc_prep(degT, x, W1):
    fo = W1.shape[1]

    def body(deg_ref, x_ref, w_ref, dis_ref, yw_ref):
        d = lax.rsqrt(deg_ref[:, 0:1] + deg_ref[:, 1:2] + 1.0)
        dis_ref[...] = d
        yw_ref[...] = d * jnp.dot(x_ref[...], w_ref[...],
                                  preferred_element_type=jnp.float32)

    return pl.pallas_call(
        body,
        grid=(N // _BN,),
        in_specs=[
            pl.BlockSpec((_BN, 2), lambda i: (i, 0)),
            pl.BlockSpec((_BN, F_IN), lambda i: (i, 0)),
            pl.BlockSpec((F_IN, fo), lambda i: (0, 0)),
        ],
        out_specs=[
            pl.BlockSpec((_BN, 1), lambda i: (i, 0)),
            pl.BlockSpec((_BN, fo), lambda i: (i, 0)),
        ],
        out_shape=[
            jax.ShapeDtypeStruct((N, 1), jnp.float32),
            jax.ShapeDtypeStruct((N, fo), jnp.float32),
        ],
    )(degT, x, W1)


def _tc_layer(acc, yw, dis, b2d, Wn):
    fi, fo = Wn.shape

    def body(acc_ref, yw_ref, dis_ref, b_ref, w_ref, out_ref):
        s = acc_ref[0] + acc_ref[1] + yw_ref[...]
        h = jnp.maximum(dis_ref[...] * s + b_ref[...], 0.0)
        out_ref[...] = dis_ref[...] * jnp.dot(
            h, w_ref[...], preferred_element_type=jnp.float32)

    return pl.pallas_call(
        body,
        grid=(N // _BN,),
        in_specs=[
            pl.BlockSpec((NC, _BN, fi), lambda i: (0, i, 0)),
            pl.BlockSpec((_BN, fi), lambda i: (i, 0)),
            pl.BlockSpec((_BN, 1), lambda i: (i, 0)),
            pl.BlockSpec((1, fi), lambda i: (0, 0)),
            pl.BlockSpec((fi, fo), lambda i: (0, 0)),
        ],
        out_specs=pl.BlockSpec((_BN, fo), lambda i: (i, 0)),
        out_shape=jax.ShapeDtypeStruct((N, fo), jnp.float32),
    )(acc, yw, dis, b2d, Wn)


def _tc_h3(acc, yw, dis, b2d):
    fi = yw.shape[1]
    bn = N

    def body(acc_ref, yw_ref, dis_ref, b_ref, out_ref):
        s = acc_ref[0] + acc_ref[1] + yw_ref[...]
        out_ref[...] = dis_ref[...] * s + b_ref[...]

    return pl.pallas_call(
        body,
        grid=(1,),
        in_specs=[
            pl.BlockSpec((NC, bn, fi), lambda i: (0, i, 0)),
            pl.BlockSpec((bn, fi), lambda i: (i, 0)),
            pl.BlockSpec((bn, 1), lambda i: (i, 0)),
            pl.BlockSpec((1, fi), lambda i: (0, 0)),
        ],
        out_specs=pl.BlockSpec((bn, fi), lambda i: (i, 0)),
        out_shape=jax.ShapeDtypeStruct((N, fi), jnp.float32),
    )(acc, yw, dis, b2d)


def _tc_encdec(h3f, enc1_W, enc1_b, enc2_W, enc2_b, dec1_W, dec1_b,
               dec2_W, dec2_b):
    K = h3f.shape[1]
    BK = 16000
    steps = K // BK

    def body(h_ref, w_ref, b1_ref, w2_ref, b2_ref, w3_ref, b3_ref,
             w4_ref, b4_ref, out_ref, acc_ref):
        i = pl.program_id(0)
        part = jnp.dot(h_ref[...], w_ref[...],
                       preferred_element_type=jnp.float32)

        @pl.when(i == 0)
        def _():
            acc_ref[...] = part

        @pl.when(i > 0)
        def _():
            acc_ref[...] = acc_ref[...] + part

        @pl.when(i == steps - 1)
        def _():
            z = jnp.maximum(acc_ref[...] + b1_ref[...], 0.0)
            z = jnp.dot(z, w2_ref[...],
                        preferred_element_type=jnp.float32) + b2_ref[...]
            z = jnp.maximum(
                jnp.dot(z, w3_ref[...],
                        preferred_element_type=jnp.float32) + b3_ref[...],
                0.0)
            t = jnp.dot(z, w4_ref[...],
                        preferred_element_type=jnp.float32) + b4_ref[...]
            out_ref[...] = 1.0 / (1.0 + jnp.exp(-t))

    return pl.pallas_call(
        body,
        grid=(steps,),
        in_specs=[
            pl.BlockSpec((1, BK), lambda i: (0, i)),
            pl.BlockSpec((BK, 128), lambda i: (i, 0)),
            pl.BlockSpec((1, 128), lambda i: (0, 0)),
            pl.BlockSpec((128, 64), lambda i: (0, 0)),
            pl.BlockSpec((1, 64), lambda i: (0, 0)),
            pl.BlockSpec((64, 128), lambda i: (0, 0)),
            pl.BlockSpec((1, 128), lambda i: (0, 0)),
            pl.BlockSpec((128, N), lambda i: (0, 0)),
            pl.BlockSpec((1, N), lambda i: (0, 0)),
        ],
        out_specs=pl.BlockSpec((1, N), lambda i: (0, 0)),
        out_shape=jax.ShapeDtypeStruct((1, N), jnp.float32),
        scratch_shapes=[pltpu.VMEM((1, 128), jnp.float32)],
    )(h3f, enc1_W, enc1_b, enc2_W, enc2_b, dec1_W, dec1_b, dec2_W, dec2_b)


def kernel(x, edge_index, batch_size, batch_index, W1, b1, W2, b2, W3, b3,
           enc1_W, enc1_b, enc2_W, enc2_b, dec1_W, dec1_b, dec2_W, dec2_b):
    del batch_size, batch_index
    pad = jnp.arange(E_PAD - E, dtype=jnp.int32) % (NPAD - N)
    padc = jnp.stack([pad, pad + N]).reshape(2, (E_PAD - E) // CHUNK, CHUNK)
    edges = jnp.concatenate(
        [edge_index.reshape(2, E // CHUNK, CHUNK), padc], axis=1)

    zzd = jnp.zeros((NODES_T,), jnp.float32)
    zz64 = jnp.zeros((NODES_T, 64), jnp.float32)
    zz32 = jnp.zeros((NODES_T, 32), jnp.float32)
    zz16 = jnp.zeros((NODES_T, 16), jnp.float32)
    ones = jnp.ones((CHUNK,), jnp.float32)

    deg = _sc_degree(edges, zzd, ones)
    dis, yw1 = _tc_prep(deg.T, x, W1)

    acc1 = _sc_propagate(yw1, edges, zz64, 64)
    yw2 = _tc_layer(acc1, yw1, dis, b1.reshape(1, -1), W2)

    acc2 = _sc_propagate(yw2, edges, zz32, 32)
    yw3 = _tc_layer(acc2, yw2, dis, b2.reshape(1, -1), W3)

    acc3 = _sc_propagate(yw3, edges, zz16, 16)
    h3 = _tc_h3(acc3, yw3, dis, b3.reshape(1, -1))

    h3f = h3.reshape(1, N * 16)
    return _tc_encdec(h3f, enc1_W, enc1_b.reshape(1, -1), enc2_W,
                      enc2_b.reshape(1, -1), dec1_W, dec1_b.reshape(1, -1),
                      dec2_W, dec2_b.reshape(1, -1))


# speedup vs baseline: 42.6179x; 1.0103x over previous
import functools

import jax
import jax.numpy as jnp
from jax import lax
from jax.experimental import pallas as pl
from jax.experimental.pallas import tpu as pltpu
from jax.experimental.pallas import tpu_sc as plsc

N = 10000
NPAD = 10240
E = 320000
F_IN = 128
NC = 2
NS = 16
NW = NC * NS
CHUNK = 128
EW = 10240
E_PAD = EW * NW
ROWS_W = EW // CHUNK
NODES_T = NPAD // NS

@functools.cache
def _mesh():
    return plsc.VectorSubcoreMesh(
        core_axis_name="c", subcore_axis_name="s",
        num_cores=NC, num_subcores=NS)


def _sc_degree(edges, zz, ones):
    K = 4

    def body(edges_hbm, zz_hbm, ones_hbm, out_hbm, idx_v, ones_v, acc_sh, sem):
        cid = lax.axis_index("c")
        sid = lax.axis_index("s")
        w = sid * NC + cid
        pltpu.sync_copy(zz_hbm, acc_sh.at[pl.ds(sid * NODES_T, NODES_T)])
        pltpu.sync_copy(ones_hbm, ones_v)
        plsc.subcore_barrier()
        pltpu.sync_copy(edges_hbm.at[1, pl.ds(w * ROWS_W, ROWS_W)], idx_v)

        def step(s, carry):
            for j in range(K):
                pltpu.async_copy(ones_v, acc_sh.at[idx_v.at[s * K + j]], sem,
                                 add=True)
            for j in range(K):
                pltpu.make_async_copy(ones_v, acc_sh.at[idx_v.at[0]],
                                      sem).wait()
            return carry

        lax.fori_loop(0, ROWS_W // K, step, 0)
        plsc.subcore_barrier()
        pltpu.sync_copy(acc_sh.at[pl.ds(sid * NODES_T, NODES_T)],
                        out_hbm.at[cid, pl.ds(sid * NODES_T, NODES_T)])

    k = pl.kernel(
        body,
        out_type=jax.ShapeDtypeStruct((NC, NPAD), jnp.float32),
        mesh=_mesh(),
        compiler_params=pltpu.CompilerParams(use_tc_tiling_on_sc=False),
        scratch_types=[
            pltpu.VMEM((ROWS_W, CHUNK), jnp.int32),
            pltpu.VMEM((CHUNK,), jnp.float32),
            pltpu.VMEM_SHARED((NPAD,), jnp.float32),
            pltpu.SemaphoreType.DMA,
        ],
    )
    return k(edges, zz, ones)


def _sc_propagate(yw, edges, zz, F):
    K = {64: 4, 32: 10, 16: 20}[F]
    NSUP = ROWS_W // K

    spmem_yw = F == 16

    def body(yw_hbm, edges_hbm, zz_hbm, out_hbm,
             sidx_v, didx_v, buf_a, buf_b, acc_sh, *rest):
        if spmem_yw:
            yw_sh, sem_ga, sem_gb, sem_s = rest
        else:
            sem_ga, sem_gb, sem_s = rest
        cid = lax.axis_index("c")
        sid = lax.axis_index("s")
        w = sid * NC + cid
        pltpu.sync_copy(zz_hbm, acc_sh.at[pl.ds(sid * NODES_T, NODES_T)])
        if spmem_yw:
            pltpu.sync_copy(yw_hbm.at[pl.ds(sid * (N // NS), N // NS)],
                            yw_sh.at[pl.ds(sid * (N // NS), N // NS)])
        plsc.subcore_barrier()
        yw_src = yw_sh if spmem_yw else yw_hbm
        pltpu.sync_copy(edges_hbm.at[0, pl.ds(w * ROWS_W, ROWS_W)], sidx_v)
        pltpu.sync_copy(edges_hbm.at[1, pl.ds(w * ROWS_W, ROWS_W)], didx_v)

        def fire(s, buf, sem):
            for j in range(K):
                pltpu.async_copy(yw_src.at[sidx_v.at[s * K + j]],
                                 buf.at[pl.ds(j * CHUNK, CHUNK)], sem)

        def drain(buf, sem):
            for j in range(K):
                pltpu.make_async_copy(yw_src.at[sidx_v.at[0]],
                                      buf.at[pl.ds(j * CHUNK, CHUNK)],
                                      sem).wait()

        def scat(s, buf):
            for j in range(K):
                pltpu.async_copy(buf.at[pl.ds(j * CHUNK, CHUNK)],
                                 acc_sh.at[didx_v.at[s * K + j]], sem_s,
                                 add=True)
            for j in range(K):
                pltpu.make_async_copy(buf.at[pl.ds(j * CHUNK, CHUNK)],
                                      acc_sh.at[didx_v.at[0]], sem_s).wait()

        fire(0, buf_a, sem_ga)

        def pair(p, carry):
            s0 = 2 * p
            s1 = s0 + 1
            drain(buf_a, sem_ga)
            fire(s1, buf_b, sem_gb)
            scat(s0, buf_a)
            drain(buf_b, sem_gb)
            fire(jnp.minimum(s1 + 1, NSUP - 1), buf_a, sem_ga)
            scat(s1, buf_b)
            return carry

        lax.fori_loop(0, NSUP // 2, pair, 0)
        drain(buf_a, sem_ga)
        plsc.subcore_barrier()
        pltpu.sync_copy(acc_sh.at[pl.ds(sid * NODES_T, NODES_T)],
                        out_hbm.at[cid, pl.ds(sid * NODES_T, NODES_T)])

    k = pl.kernel(
        body,
        out_type=jax.ShapeDtypeStruct((NC, NPAD, F), jnp.float32),
        mesh=_mesh(),
        compiler_params=pltpu.CompilerParams(use_tc_tiling_on_sc=False),
        scratch_types=[
            pltpu.VMEM((ROWS_W, CHUNK), jnp.int32),
            pltpu.VMEM((ROWS_W, CHUNK), jnp.int32),
            pltpu.VMEM((K * CHUNK, F), jnp.float32),
            pltpu.VMEM((K * CHUNK, F), jnp.float32),
            pltpu.VMEM_SHARED((NPAD, F), jnp.float32),
        ] + ([pltpu.VMEM_SHARED((NPAD, F), jnp.float32)] if spmem_yw else [])
        + [
            pltpu.SemaphoreType.DMA,
            pltpu.SemaphoreType.DMA,
            pltpu.SemaphoreType.DMA,
        ],
    )
    return k(yw, edges, zz)


_BN = 2000


def _tc_prep(degT, x, W1):
    fo = W1.shape[1]

    def body(deg_ref, x_ref, w_ref, dis_ref, yw_ref):
        d = lax.rsqrt(deg_ref[:, 0:1] + deg_ref[:, 1:2] + 1.0)
        dis_ref[...] = d
        yw_ref[...] = d * jnp.dot(x_ref[...], w_ref[...],
                                  preferred_element_type=jnp.float32)

    return pl.pallas_call(
        body,
        grid=(N // _BN,),
        in_specs=[
            pl.BlockSpec((_BN, 2), lambda i: (i, 0)),
            pl.BlockSpec((_BN, F_IN), lambda i: (i, 0)),
            pl.BlockSpec((F_IN, fo), lambda i: (0, 0)),
        ],
        out_specs=[
            pl.BlockSpec((_BN, 1), lambda i: (i, 0)),
            pl.BlockSpec((_BN, fo), lambda i: (i, 0)),
        ],
        out_shape=[
            jax.ShapeDtypeStruct((N, 1), jnp.float32),
            jax.ShapeDtypeStruct((N, fo), jnp.float32),
        ],
    )(degT, x, W1)


def _tc_layer(acc, yw, dis, b2d, Wn):
    fi, fo = Wn.shape

    def body(acc_ref, yw_ref, dis_ref, b_ref, w_ref, out_ref):
        s = acc_ref[0] + acc_ref[1] + yw_ref[...]
        h = jnp.maximum(dis_ref[...] * s + b_ref[...], 0.0)
        out_ref[...] = dis_ref[...] * jnp.dot(
            h, w_ref[...], preferred_element_type=jnp.float32)

    return pl.pallas_call(
        body,
        grid=(N // _BN,),
        in_specs=[
            pl.BlockSpec((NC, _BN, fi), lambda i: (0, i, 0)),
            pl.BlockSpec((_BN, fi), lambda i: (i, 0)),
            pl.BlockSpec((_BN, 1), lambda i: (i, 0)),
            pl.BlockSpec((1, fi), lambda i: (0, 0)),
            pl.BlockSpec((fi, fo), lambda i: (0, 0)),
        ],
        out_specs=pl.BlockSpec((_BN, fo), lambda i: (i, 0)),
        out_shape=jax.ShapeDtypeStruct((N, fo), jnp.float32),
    )(acc, yw, dis, b2d, Wn)


def _tc_h3(acc, yw, dis, b2d):
    fi = yw.shape[1]
    bn = N

    def body(acc_ref, yw_ref, dis_ref, b_ref, out_ref):
        s = acc_ref[0] + acc_ref[1] + yw_ref[...]
        out_ref[...] = dis_ref[...] * s + b_ref[...]

    return pl.pallas_call(
        body,
        grid=(1,),
        in_specs=[
            pl.BlockSpec((NC, bn, fi), lambda i: (0, i, 0)),
            pl.BlockSpec((bn, fi), lambda i: (i, 0)),
            pl.BlockSpec((bn, 1), lambda i: (i, 0)),
            pl.BlockSpec((1, fi), lambda i: (0, 0)),
        ],
        out_specs=pl.BlockSpec((bn, fi), lambda i: (i, 0)),
        out_shape=jax.ShapeDtypeStruct((N, fi), jnp.float32),
    )(acc, yw, dis, b2d)


def _tc_encdec(h3f, enc1_W, enc1_b, enc2_W, enc2_b, dec1_W, dec1_b,
               dec2_W, dec2_b):
    K = h3f.shape[1]
    BK = 16000
    steps = K // BK

    def body(h_ref, w_ref, b1_ref, w2_ref, b2_ref, w3_ref, b3_ref,
             w4_ref, b4_ref, out_ref, acc_ref):
        i = pl.program_id(0)
        part = jnp.dot(h_ref[...], w_ref[...],
                       preferred_element_type=jnp.float32)

        @pl.when(i == 0)
        def _():
            acc_ref[...] = part

        @pl.when(i > 0)
        def _():
            acc_ref[...] = acc_ref[...] + part

        @pl.when(i == steps - 1)
        def _():
            z = jnp.maximum(acc_ref[...] + b1_ref[...], 0.0)
            z = jnp.dot(z, w2_ref[...],
                        preferred_element_type=jnp.float32) + b2_ref[...]
            z = jnp.maximum(
                jnp.dot(z, w3_ref[...],
                        preferred_element_type=jnp.float32) + b3_ref[...],
                0.0)
            t = jnp.dot(z, w4_ref[...],
                        preferred_element_type=jnp.float32) + b4_ref[...]
            out_ref[...] = 1.0 / (1.0 + jnp.exp(-t))

    return pl.pallas_call(
        body,
        grid=(steps,),
        in_specs=[
            pl.BlockSpec((1, BK), lambda i: (0, i)),
            pl.BlockSpec((BK, 128), lambda i: (i, 0)),
            pl.BlockSpec((1, 128), lambda i: (0, 0)),
            pl.BlockSpec((128, 64), lambda i: (0, 0)),
            pl.BlockSpec((1, 64), lambda i: (0, 0)),
            pl.BlockSpec((64, 128), lambda i: (0, 0)),
            pl.BlockSpec((1, 128), lambda i: (0, 0)),
            pl.BlockSpec((128, N), lambda i: (0, 0)),
            pl.BlockSpec((1, N), lambda i: (0, 0)),
        ],
        out_specs=pl.BlockSpec((1, N), lambda i: (0, 0)),
        out_shape=jax.ShapeDtypeStruct((1, N), jnp.float32),
        scratch_shapes=[pltpu.VMEM((1, 128), jnp.float32)],
    )(h3f, enc1_W, enc1_b, enc2_W, enc2_b, dec1_W, dec1_b, dec2_W, dec2_b)


def kernel(x, edge_index, batch_size, batch_index, W1, b1, W2, b2, W3, b3,
           enc1_W, enc1_b, enc2_W, enc2_b, dec1_W, dec1_b, dec2_W, dec2_b):
    del batch_size, batch_index
    pad = jnp.arange(E_PAD - E, dtype=jnp.int32) % (NPAD - N)
    padc = jnp.stack([pad, pad + N]).reshape(2, (E_PAD - E) // CHUNK, CHUNK)
    edges = jnp.concatenate(
        [edge_index.reshape(2, E // CHUNK, CHUNK), padc], axis=1)

    zzd = jnp.zeros((NODES_T,), jnp.float32)
    zz64 = jnp.zeros((NODES_T, 64), jnp.float32)
    zz32 = jnp.zeros((NODES_T, 32), jnp.float32)
    zz16 = jnp.zeros((NODES_T, 16), jnp.float32)
    ones = jnp.ones((CHUNK,), jnp.float32)

    deg = _sc_degree(edges, zzd, ones)
    dis, yw1 = _tc_prep(deg.T, x, W1)

    acc1 = _sc_propagate(yw1, edges, zz64, 64)
    yw2 = _tc_layer(acc1, yw1, dis, b1.reshape(1, -1), W2)

    acc2 = _sc_propagate(yw2, edges, zz32, 32)
    yw3 = _tc_layer(acc2, yw2, dis, b2.reshape(1, -1), W3)

    acc3 = _sc_propagate(yw3, edges, zz16, 16)
    h3 = _tc_h3(acc3, yw3, dis, b3.reshape(1, -1))

    h3f = h3.reshape(1, N * 16)
    return _tc_encdec(h3f, enc1_W, enc1_b.reshape(1, -1), enc2_W,
                      enc2_b.reshape(1, -1), dec1_W, dec1_b.reshape(1, -1),
                      dec2_W, dec2_b.reshape(1, -1))
